# speedup vs baseline: 1.0001x; 1.0001x over previous
import jax
import jax.numpy as jnp
from jax.experimental import pallas as pl


def _seg_softmax(alpha, seg, num_segments):
    amax = jax.ops.segment_max(alpha, seg, num_segments=num_segments)
    amax = jnp.where(jnp.isfinite(amax), amax, 0.0)
    ex = jnp.exp(alpha - amax[seg])
    s = jax.ops.segment_sum(ex, seg, num_segments=num_segments)
    return ex / (s[seg] + 1e-16)


def _gat_conv(x, edge_index, W, att_s, att_d, bias, H, C):
    N = x.shape[0]
    h = (x @ W).reshape(N, H, C)
    src = edge_index[0]
    dst = edge_index[1]
    a_s = jnp.sum(h * att_s, axis=-1)
    a_d = jnp.sum(h * att_d, axis=-1)
    e = jax.nn.leaky_relu(a_s[src] + a_d[dst], negative_slope=0.2)
    alpha = _seg_softmax(e, dst, N)
    msg = h[src] * alpha[:, :, None]
    out = jax.ops.segment_sum(msg, dst, num_segments=N)
    return out.reshape(N, H * C) + bias


def _batchnorm(x, w, b, eps=1e-5):
    mu = jnp.mean(x, axis=0)
    var = jnp.mean((x - mu) ** 2, axis=0)
    return (x - mu) / jnp.sqrt(var + eps) * w + b


def _fc_body(x_ref, w_ref, b_ref, o_ref):
    o_ref[...] = (
        jnp.dot(x_ref[...], w_ref[...], preferred_element_type=jnp.float32)
        + b_ref[...]
    )


def _fc(pooled, Wfc, bfc):
    return pl.pallas_call(
        _fc_body,
        out_shape=jax.ShapeDtypeStruct((pooled.shape[0], Wfc.shape[1]), jnp.float32),
    )(pooled, Wfc, bfc.reshape(1, -1))


def kernel(x, edge_index, batch, W1, as1, ad1, b1, g1, be1, W2, as2, ad2, b2, g2,
           be2, W3, as3, ad3, b3, g3, be3, Wfc, bfc):
    h = jax.nn.relu(_batchnorm(_gat_conv(x, edge_index, W1, as1, ad1, b1, 8, 16), g1, be1))
    h = jax.nn.relu(_batchnorm(_gat_conv(h, edge_index, W2, as2, ad2, b2, 8, 32), g2, be2))
    h = jax.nn.relu(_batchnorm(_gat_conv(h, edge_index, W3, as3, ad3, b3, 8, 64), g3, be3))
    pooled = jax.ops.segment_max(h, batch, num_segments=64)
    pooled = jnp.where(jnp.isfinite(pooled), pooled, 0.0)
    return _fc(pooled, Wfc, bfc)


# speedup vs baseline: 16.7510x; 16.7497x over previous
import functools

import jax
import jax.numpy as jnp
from jax import lax
from jax.experimental import pallas as pl
from jax.experimental.pallas import tpu as pltpu
from jax.experimental.pallas import tpu_sc as plsc

NC, NS, L = 2, 16, 16


def _mesh():
    return plsc.VectorSubcoreMesh(
        core_axis_name="c", subcore_axis_name="s", num_cores=NC, num_subcores=NS
    )


_SC_PARAMS = pltpu.CompilerParams(
    use_tc_tiling_on_sc=False, needs_layout_passes=False
)


def _dense_body(x_ref, w_ref, aw_ref, *rest, bn, K, n):
    if bn:
        mom_ref, g_ref, be_ref, h_ref, as_ref, ad_ref = rest
        mu = mom_ref[0:1, :] / n
        var = mom_ref[1:2, :] / n - mu * mu
        xin = (x_ref[...] - mu) * lax.rsqrt(var + 1e-5) * g_ref[...] + be_ref[...]
        xin = jnp.maximum(xin, 0.0)
    else:
        h_ref, as_ref, ad_ref = rest
        xin = x_ref[...]
    h = jnp.dot(xin, w_ref[...], preferred_element_type=jnp.float32)
    Wc = h_ref.shape[2]
    for k in range(K):
        h_ref[k] = h[:, k * Wc:(k + 1) * Wc]
    asd = jnp.dot(h, aw_ref[...], preferred_element_type=jnp.float32)
    as_ref[...] = asd[:, 0:16]
    ad_ref[...] = asd[:, 16:32]


def _dense(xin, W, aw, mom, g, be, K, bn):
    N, Fin = xin.shape
    F = W.shape[1]
    Wc = F // K
    R = 1000
    body = functools.partial(_dense_body, bn=bn, K=K, n=N)
    in_specs = [
        pl.BlockSpec((R, Fin), lambda i: (i, 0)),
        pl.BlockSpec((Fin, F), lambda i: (0, 0)),
        pl.BlockSpec((F, 32), lambda i: (0, 0)),
    ]
    args = [xin, W, aw]
    if bn:
        in_specs += [
            pl.BlockSpec((2, Fin), lambda i: (0, 0)),
            pl.BlockSpec((1, Fin), lambda i: (0, 0)),
            pl.BlockSpec((1, Fin), lambda i: (0, 0)),
        ]
        args += [mom, g.reshape(1, Fin), be.reshape(1, Fin)]
    return pl.pallas_call(
        body,
        grid=(N // R,),
        in_specs=in_specs,
        out_specs=[
            pl.BlockSpec((K, R, Wc), lambda i: (0, i, 0)),
            pl.BlockSpec((R, 16), lambda i: (i, 0)),
            pl.BlockSpec((R, 16), lambda i: (i, 0)),
        ],
        out_shape=[
            jax.ShapeDtypeStruct((K, N, Wc), jnp.float32),
            jax.ShapeDtypeStruct((N, 16), jnp.float32),
            jax.ShapeDtypeStruct((N, 16), jnp.float32),
        ],
    )(*args)


def _post_body(raw_ref, s_ref, b_ref, y_ref, mom_ref, *, K, C):
    @pl.when(pl.program_id(0) == 0)
    def _():
        mom_ref[...] = jnp.zeros_like(mom_ref)

    raw = jnp.concatenate([raw_ref[k] for k in range(K)], axis=1)
    sinv = 1.0 / (s_ref[...][:, 0:8] + 1e-16)
    ys = [raw[:, hh * C:(hh + 1) * C] * sinv[:, hh:hh + 1] for hh in range(8)]
    y = jnp.concatenate(ys, axis=1) + b_ref[...]
    y_ref[...] = y
    mom_ref[0:1, :] += jnp.sum(y, axis=0, keepdims=True)
    mom_ref[1:2, :] += jnp.sum(y * y, axis=0, keepdims=True)


def _post(raw, s, b, K, C):
    _, N, Wc = raw.shape
    F = K * Wc
    R = 1000
    return pl.pallas_call(
        functools.partial(_post_body, K=K, C=C),
        grid=(N // R,),
        in_specs=[
            pl.BlockSpec((K, R, Wc), lambda i: (0, i, 0)),
            pl.BlockSpec((R, 16), lambda i: (i, 0)),
            pl.BlockSpec((1, F), lambda i: (0, 0)),
        ],
        out_specs=[
            pl.BlockSpec((R, F), lambda i: (i, 0)),
            pl.BlockSpec((2, F), lambda i: (0, 0)),
        ],
        out_shape=[
            jax.ShapeDtypeStruct((N, F), jnp.float32),
            jax.ShapeDtypeStruct((2, F), jnp.float32),
        ],
    )(raw, s, b.reshape(1, F))


def _pool_body(y_ref, bid_ref, mom_ref, g_ref, be_ref, out_ref, *, n):
    @pl.when(pl.program_id(0) == 0)
    def _():
        out_ref[...] = jnp.full_like(out_ref[...], -jnp.inf)

    mu = mom_ref[0:1, :] / n
    var = mom_ref[1:2, :] / n - mu * mu
    y = (y_ref[...] - mu) * lax.rsqrt(var + 1e-5) * g_ref[...] + be_ref[...]
    y = jnp.maximum(y, 0.0)
    bid = bid_ref[...]
    lo = jnp.min(bid)
    hi = jnp.max(bid)

    def body(b, _):
        m = bid == b
        cm = jnp.max(jnp.where(m, y, -jnp.inf), axis=0, keepdims=True)
        out_ref[pl.ds(b, 1), :] = jnp.maximum(out_ref[pl.ds(b, 1), :], cm)
        return 0

    lax.fori_loop(lo, hi + 1, body, 0)


def _pool(y, bid, mom, g, be):
    N, F = y.shape
    R = 200
    return pl.pallas_call(
        functools.partial(_pool_body, n=N),
        grid=(N // R,),
        in_specs=[
            pl.BlockSpec((R, F), lambda i: (i, 0)),
            pl.BlockSpec((R, 1), lambda i: (i, 0)),
            pl.BlockSpec((2, F), lambda i: (0, 0)),
            pl.BlockSpec((1, F), lambda i: (0, 0)),
            pl.BlockSpec((1, F), lambda i: (0, 0)),
        ],
        out_specs=pl.BlockSpec((64, F), lambda i: (0, 0)),
        out_shape=jax.ShapeDtypeStruct((64, F), jnp.float32),
    )(y, bid, mom, g.reshape(1, F), be.reshape(1, F))


def _fc_body(p_ref, w_ref, b_ref, o_ref):
    p = p_ref[...]
    p = jnp.where(jnp.isfinite(p), p, 0.0)
    o_ref[...] = jnp.dot(p, w_ref[...], preferred_element_type=jnp.float32) + b_ref[...]


def _fc(pooled, Wfc, bfc):
    return pl.pallas_call(
        _fc_body,
        out_shape=jax.ShapeDtypeStruct((pooled.shape[0], Wfc.shape[1]), jnp.float32),
    )(pooled, Wfc, bfc.reshape(1, -1))


def _edge(src, dst, asd_s, asd_d):
    E = src.shape[0]
    B = 128
    nblk = E // B
    base_cnt = nblk // (NC * NS)
    rem = nblk % (NC * NS)

    @functools.partial(
        pl.kernel,
        out_type=[
            jax.ShapeDtypeStruct((E * 8,), jnp.float32),
            jax.ShapeDtypeStruct((2, 16), jnp.float32),
        ],
        mesh=_mesh(),
        scratch_types=[
            pltpu.VMEM((B,), jnp.int32),
            pltpu.VMEM((B,), jnp.int32),
            pltpu.VMEM((B, 16), jnp.float32),
            pltpu.VMEM((B, 16), jnp.float32),
            pltpu.VMEM((B * 8 + 16,), jnp.float32),
            pltpu.VMEM((NS, 16), jnp.float32),
            pltpu.VMEM((16,), jnp.float32),
            pltpu.VMEM_SHARED((NS, 16), jnp.float32),
            pltpu.SemaphoreType.DMA,
            pltpu.SemaphoreType.DMA,
        ],
        compiler_params=_SC_PARAMS,
    )
    def k(src_h, dst_h, as_h, ad_h, e_h, m2_h, src_v, dst_v, srows, drows, eblk,
          red, mxb, shmax, sem1, sem2):
        c = lax.axis_index("c")
        s = lax.axis_index("s")
        wid = c * NS + s
        lane = lax.iota(jnp.int32, L)
        neg = jnp.zeros((L,), jnp.float32) - jnp.inf
        cnt = base_cnt + jnp.where(wid < rem, 1, 0)

        def blk(jb, mx):
            off = pl.multiple_of((wid + jb * (NC * NS)) * B, 8)
            pltpu.sync_copy(src_h.at[pl.ds(off, B)], src_v)
            pltpu.sync_copy(dst_h.at[pl.ds(off, B)], dst_v)
            cp1 = pltpu.async_copy(as_h.at[src_v], srows, sem1)
            cp2 = pltpu.async_copy(ad_h.at[dst_v], drows, sem2)
            cp1.wait()
            cp2.wait()

            def edge(r, mx):
                v = srows[r] + drows[r]
                e16 = jnp.maximum(v, 0.2 * v)
                eblk[pl.ds(r * 8, 16)] = e16
                return jnp.maximum(mx, jnp.where(lane < 8, e16, neg))

            mx = lax.fori_loop(0, B, edge, mx)
            pltpu.sync_copy(eblk.at[pl.ds(0, B * 8)], e_h.at[pl.ds(off * 8, B * 8)])
            return mx

        mx = lax.fori_loop(0, cnt, blk, neg)
        mxb[...] = mx
        pltpu.sync_copy(mxb, shmax.at[s])
        plsc.subcore_barrier()

        @pl.when(s == 0)
        def _():
            pltpu.sync_copy(shmax, red)
            m = red[0]
            for t in range(1, NS):
                m = jnp.maximum(m, red[t])
            mxb[...] = m
            pltpu.sync_copy(mxb, m2_h.at[c])

    return k(src, dst, asd_s, asd_d)


def _agg(src, dst, h3, e1, m2, N, K, C):
    E = src.shape[0]
    Wc = h3.shape[1]
    hpc = Wc // C
    Kp = K // 2
    B = 128
    nblk = E // B
    base_cnt = nblk // NS
    rem = nblk % NS
    RPT = N // NS
    zW = jnp.zeros((RPT, Wc), jnp.float32)
    zS = jnp.zeros((RPT, 16), jnp.float32)

    @functools.partial(
        pl.kernel,
        out_type=[
            jax.ShapeDtypeStruct((K * N, Wc), jnp.float32),
            jax.ShapeDtypeStruct((N, 16), jnp.float32),
        ],
        mesh=_mesh(),
        scratch_types=[
            pltpu.VMEM((B,), jnp.int32),
            pltpu.VMEM((B,), jnp.int32),
            pltpu.VMEM((B,), jnp.int32),
            pltpu.VMEM((B, Wc), jnp.float32),
            pltpu.VMEM((B * 8 + 16,), jnp.float32),
            pltpu.VMEM((B * 8 + 16,), jnp.float32),
            pltpu.VMEM((B, 16), jnp.float32),
            pltpu.VMEM((2, 16), jnp.float32),
            pltpu.VMEM((16,), jnp.float32),
            pltpu.VMEM_SHARED((N, Wc), jnp.float32),
            pltpu.VMEM_SHARED((N, 16), jnp.float32),
            pltpu.SemaphoreType.DMA,
        ],
        compiler_params=_SC_PARAMS,
    )
    def k(src_h, dst_h, h3_h, e_h, m2_h, zW_h, zS_h, raw_h, s_h, src_v, dst_v,
          idx_v, rows, ev, exv, exb, m2v, mbuf, acc, sacc, sem):
        c = lax.axis_index("c")
        s = lax.axis_index("s")
        lane = lax.iota(jnp.int32, L)
        pltpu.sync_copy(m2_h, m2v)
        mv16 = jnp.maximum(m2v[0], m2v[1])
        l8 = lane & 7
        mpat = jnp.zeros((L,), jnp.float32)
        for hh in range(8):
            mpat = jnp.where(l8 == hh, mv16[hh], mpat)

        for kk in range(Kp):
            chunk = c * Kp + kk
            pltpu.sync_copy(zW_h, acc.at[pl.ds(s * RPT, RPT)])
            if kk == 0:
                @pl.when(c == 0)
                def _():
                    pltpu.sync_copy(zS_h, sacc.at[pl.ds(s * RPT, RPT)])
            plsc.subcore_barrier()
            cnt = base_cnt + jnp.where(s < rem, 1, 0)

            def blk(jb, _):
                off = pl.multiple_of((s + jb * NS) * B, 8)
                pltpu.sync_copy(src_h.at[pl.ds(off, B)], src_v)
                pltpu.sync_copy(dst_h.at[pl.ds(off, B)], dst_v)
                pltpu.sync_copy(e_h.at[pl.ds(off * 8, B * 8)], ev.at[pl.ds(0, B * 8)])

                def idxb(i, _):
                    idx_v[pl.ds(i * L, L)] = src_v[pl.ds(i * L, L)] + chunk * N
                    return 0

                lax.fori_loop(0, B // L, idxb, 0)
                pltpu.async_copy(h3_h.at[idx_v], rows, sem).wait()

                def exps(i, _):
                    v = ev[pl.ds(i * L, L)]
                    exv[pl.ds(i * L, L)] = jnp.exp(v - mpat)
                    return 0

                lax.fori_loop(0, B * 8 // L, exps, 0)

                def scale(r, _):
                    rsp = jnp.zeros((L,), jnp.int32) + r
                    v8 = exv[pl.ds(r * 8, 16)]
                    for hh in range(hpc):
                        exs = jnp.sum(jnp.where(lane == chunk * hpc + hh, v8, 0.0))
                        for j in range(C // L):
                            cv = lane + (hh * C + j * L)
                            v = plsc.load_gather(rows, [rsp, cv])
                            plsc.store_scatter(rows, [rsp, cv], v * exs)
                    return 0

                lax.fori_loop(0, B, scale, 0)

                if kk == 0:
                    @pl.when(c == 0)
                    def _():
                        def srow(r, _):
                            rsp = jnp.zeros((L,), jnp.int32) + r
                            v = exv[pl.ds(r * 8, 16)]
                            plsc.store_scatter(
                                exb, [rsp, lane],
                                jnp.where(lane < 8, v, jnp.zeros((L,), jnp.float32)))
                            return 0

                        lax.fori_loop(0, B, srow, 0)
                        pltpu.sync_copy(exb, sacc.at[dst_v], add=True)

                pltpu.sync_copy(rows, acc.at[dst_v], add=True)
                return 0

            lax.fori_loop(0, cnt, blk, 0)
            plsc.subcore_barrier()
            pltpu.sync_copy(acc.at[pl.ds(s * RPT, RPT)],
                            raw_h.at[pl.ds(chunk * N + s * RPT, RPT)])
            if kk == 0:
                @pl.when(c == 0)
                def _():
                    pltpu.sync_copy(sacc.at[pl.ds(s * RPT, RPT)],
                                    s_h.at[pl.ds(s * RPT, RPT)])
            plsc.subcore_barrier()

    return k(src, dst, h3, e1, m2, zW, zS)


def _att_w(att_s, att_d, F, C):
    hid = jnp.repeat(jnp.arange(8), C)
    onehot = (hid[:, None] == jnp.arange(8)[None, :]).astype(jnp.float32)
    AS = att_s.reshape(F, 1) * onehot
    AD = att_d.reshape(F, 1) * onehot
    z = jnp.zeros((F, 8), jnp.float32)
    return jnp.concatenate([AS, z, AD, z], axis=1)


def _layer(src, dst, xin, W, aw, mom, g, be, b, K, C, bn):
    N = xin.shape[0]
    h3, as_t, ad_t = _dense(xin, W, aw, mom, g, be, K=K, bn=bn)
    Wc = h3.shape[2]
    e1, m2 = _edge(src, dst, as_t, ad_t)
    raw, s = _agg(src, dst, h3.reshape(K * N, Wc), e1, m2, N, K, C)
    return _post(raw.reshape(K, N, Wc), s, b, K=K, C=C)


def kernel(x, edge_index, batch, W1, as1, ad1, b1, g1, be1, W2, as2, ad2, b2, g2,
           be2, W3, as3, ad3, b3, g3, be3, Wfc, bfc):
    N = x.shape[0]
    src = edge_index[0]
    dst = edge_index[1]
    xp = jnp.pad(x, ((0, 0), (0, 5)))
    W1p = jnp.pad(W1, ((0, 5), (0, 0)))
    aw1 = _att_w(as1, ad1, 128, 16)
    aw2 = _att_w(as2, ad2, 256, 32)
    aw3 = _att_w(as3, ad3, 512, 64)

    y1, mom1 = _layer(src, dst, xp, W1p, aw1, None, None, None, b1, K=2, C=16, bn=False)
    y2, mom2 = _layer(src, dst, y1, W2, aw2, mom1, g1, be1, b2, K=2, C=32, bn=True)
    y3, mom3 = _layer(src, dst, y2, W3, aw3, mom2, g2, be2, b3, K=4, C=64, bn=True)

    pooled = _pool(y3, batch.reshape(N, 1), mom3, g3, be3)
    return _fc(pooled, Wfc, bfc)


# speedup vs baseline: 23.1578x; 1.3825x over previous
import functools

import jax
import jax.numpy as jnp
from jax import lax
from jax.experimental import pallas as pl
from jax.experimental.pallas import tpu as pltpu
from jax.experimental.pallas import tpu_sc as plsc

NC, NS, L = 2, 16, 16


def _mesh():
    return plsc.VectorSubcoreMesh(
        core_axis_name="c", subcore_axis_name="s", num_cores=NC, num_subcores=NS
    )


_SC_PARAMS = pltpu.CompilerParams(
    use_tc_tiling_on_sc=False, needs_layout_passes=False
)


def _dense_body(x_ref, w_ref, aw_ref, *rest, bn, K, n):
    if bn:
        mom_ref, g_ref, be_ref, h_ref, as_ref, ad_ref = rest
        mu = mom_ref[0:1, :] / n
        var = mom_ref[1:2, :] / n - mu * mu
        xin = (x_ref[...] - mu) * lax.rsqrt(var + 1e-5) * g_ref[...] + be_ref[...]
        xin = jnp.maximum(xin, 0.0)
    else:
        h_ref, as_ref, ad_ref = rest
        xin = x_ref[...]
    h = jnp.dot(xin, w_ref[...], preferred_element_type=jnp.float32)
    Wc = h_ref.shape[2]
    for k in range(K):
        h_ref[k] = h[:, k * Wc:(k + 1) * Wc]
    asd = jnp.dot(h, aw_ref[...], preferred_element_type=jnp.float32)
    as_ref[...] = asd[:, 0:16]
    ad_ref[...] = asd[:, 16:32]


def _dense(xin, W, aw, mom, g, be, K, bn):
    N, Fin = xin.shape
    F = W.shape[1]
    Wc = F // K
    R = 1000
    body = functools.partial(_dense_body, bn=bn, K=K, n=N)
    in_specs = [
        pl.BlockSpec((R, Fin), lambda i: (i, 0)),
        pl.BlockSpec((Fin, F), lambda i: (0, 0)),
        pl.BlockSpec((F, 32), lambda i: (0, 0)),
    ]
    args = [xin, W, aw]
    if bn:
        in_specs += [
            pl.BlockSpec((2, Fin), lambda i: (0, 0)),
            pl.BlockSpec((1, Fin), lambda i: (0, 0)),
            pl.BlockSpec((1, Fin), lambda i: (0, 0)),
        ]
        args += [mom, g.reshape(1, Fin), be.reshape(1, Fin)]
    return pl.pallas_call(
        body,
        grid=(N // R,),
        in_specs=in_specs,
        out_specs=[
            pl.BlockSpec((K, R, Wc), lambda i: (0, i, 0)),
            pl.BlockSpec((R, 16), lambda i: (i, 0)),
            pl.BlockSpec((R, 16), lambda i: (i, 0)),
        ],
        out_shape=[
            jax.ShapeDtypeStruct((K, N, Wc), jnp.float32),
            jax.ShapeDtypeStruct((N, 16), jnp.float32),
            jax.ShapeDtypeStruct((N, 16), jnp.float32),
        ],
    )(*args)


def _post_body(raw_ref, s_ref, b_ref, y_ref, mom_ref, *, K, C):
    @pl.when(pl.program_id(0) == 0)
    def _():
        mom_ref[...] = jnp.zeros_like(mom_ref)

    raw = jnp.concatenate([raw_ref[k] for k in range(K)], axis=1)
    sinv = 1.0 / (s_ref[...][:, 0:8] + 1e-16)
    ys = [raw[:, hh * C:(hh + 1) * C] * sinv[:, hh:hh + 1] for hh in range(8)]
    y = jnp.concatenate(ys, axis=1) + b_ref[...]
    y_ref[...] = y
    mom_ref[0:1, :] += jnp.sum(y, axis=0, keepdims=True)
    mom_ref[1:2, :] += jnp.sum(y * y, axis=0, keepdims=True)


def _post(raw, s, b, K, C):
    _, N, Wc = raw.shape
    F = K * Wc
    R = 1000
    return pl.pallas_call(
        functools.partial(_post_body, K=K, C=C),
        grid=(N // R,),
        in_specs=[
            pl.BlockSpec((K, R, Wc), lambda i: (0, i, 0)),
            pl.BlockSpec((R, 16), lambda i: (i, 0)),
            pl.BlockSpec((1, F), lambda i: (0, 0)),
        ],
        out_specs=[
            pl.BlockSpec((R, F), lambda i: (i, 0)),
            pl.BlockSpec((2, F), lambda i: (0, 0)),
        ],
        out_shape=[
            jax.ShapeDtypeStruct((N, F), jnp.float32),
            jax.ShapeDtypeStruct((2, F), jnp.float32),
        ],
    )(raw, s, b.reshape(1, F))


def _pool_body(y_ref, bid_ref, mom_ref, g_ref, be_ref, out_ref, *, n):
    @pl.when(pl.program_id(0) == 0)
    def _():
        out_ref[...] = jnp.full_like(out_ref[...], -jnp.inf)

    mu = mom_ref[0:1, :] / n
    var = mom_ref[1:2, :] / n - mu * mu
    y = (y_ref[...] - mu) * lax.rsqrt(var + 1e-5) * g_ref[...] + be_ref[...]
    y = jnp.maximum(y, 0.0)
    bid = bid_ref[...]
    lo = jnp.min(bid)
    hi = jnp.max(bid)

    def body(b, _):
        m = bid == b
        cm = jnp.max(jnp.where(m, y, -jnp.inf), axis=0, keepdims=True)
        out_ref[pl.ds(b, 1), :] = jnp.maximum(out_ref[pl.ds(b, 1), :], cm)
        return 0

    lax.fori_loop(lo, hi + 1, body, 0)


def _pool(y, bid, mom, g, be):
    N, F = y.shape
    R = 200
    return pl.pallas_call(
        functools.partial(_pool_body, n=N),
        grid=(N // R,),
        in_specs=[
            pl.BlockSpec((R, F), lambda i: (i, 0)),
            pl.BlockSpec((R, 1), lambda i: (i, 0)),
            pl.BlockSpec((2, F), lambda i: (0, 0)),
            pl.BlockSpec((1, F), lambda i: (0, 0)),
            pl.BlockSpec((1, F), lambda i: (0, 0)),
        ],
        out_specs=pl.BlockSpec((64, F), lambda i: (0, 0)),
        out_shape=jax.ShapeDtypeStruct((64, F), jnp.float32),
    )(y, bid, mom, g.reshape(1, F), be.reshape(1, F))


def _fc_body(p_ref, w_ref, b_ref, o_ref):
    p = p_ref[...]
    p = jnp.where(jnp.isfinite(p), p, 0.0)
    o_ref[...] = jnp.dot(p, w_ref[...], preferred_element_type=jnp.float32) + b_ref[...]


def _fc(pooled, Wfc, bfc):
    return pl.pallas_call(
        _fc_body,
        out_shape=jax.ShapeDtypeStruct((pooled.shape[0], Wfc.shape[1]), jnp.float32),
    )(pooled, Wfc, bfc.reshape(1, -1))


def _edge(src, dst, asd_s, asd_d):
    E = src.shape[0]
    B = 128
    nblk = E // B
    base_cnt = nblk // (NC * NS)
    rem = nblk % (NC * NS)

    @functools.partial(
        pl.kernel,
        out_type=[
            jax.ShapeDtypeStruct((E * 8,), jnp.float32),
            jax.ShapeDtypeStruct((2, 16), jnp.float32),
        ],
        mesh=_mesh(),
        scratch_types=[
            pltpu.VMEM((B,), jnp.int32),
            pltpu.VMEM((B,), jnp.int32),
            pltpu.VMEM((B, 16), jnp.float32),
            pltpu.VMEM((B, 16), jnp.float32),
            pltpu.VMEM((B * 8 + 16,), jnp.float32),
            pltpu.VMEM((NS, 16), jnp.float32),
            pltpu.VMEM((16,), jnp.float32),
            pltpu.VMEM_SHARED((NS, 16), jnp.float32),
            pltpu.SemaphoreType.DMA,
            pltpu.SemaphoreType.DMA,
        ],
        compiler_params=_SC_PARAMS,
    )
    def k(src_h, dst_h, as_h, ad_h, e_h, m2_h, src_v, dst_v, srows, drows, eblk,
          red, mxb, shmax, sem1, sem2):
        c = lax.axis_index("c")
        s = lax.axis_index("s")
        wid = c * NS + s
        lane = lax.iota(jnp.int32, L)
        neg = jnp.zeros((L,), jnp.float32) - jnp.inf
        cnt = base_cnt + jnp.where(wid < rem, 1, 0)

        def blk(jb, mx):
            off = pl.multiple_of((wid + jb * (NC * NS)) * B, 8)
            pltpu.sync_copy(src_h.at[pl.ds(off, B)], src_v)
            pltpu.sync_copy(dst_h.at[pl.ds(off, B)], dst_v)
            cp1 = pltpu.async_copy(as_h.at[src_v], srows, sem1)
            cp2 = pltpu.async_copy(ad_h.at[dst_v], drows, sem2)
            cp1.wait()
            cp2.wait()

            def edge(r, mx):
                v = srows[r] + drows[r]
                e16 = jnp.maximum(v, 0.2 * v)
                eblk[pl.ds(r * 8, 16)] = e16
                return jnp.maximum(mx, jnp.where(lane < 8, e16, neg))

            mx = lax.fori_loop(0, B, edge, mx)
            pltpu.sync_copy(eblk.at[pl.ds(0, B * 8)], e_h.at[pl.ds(off * 8, B * 8)])
            return mx

        mx = lax.fori_loop(0, cnt, blk, neg)
        mxb[...] = mx
        pltpu.sync_copy(mxb, shmax.at[s])
        plsc.subcore_barrier()

        @pl.when(s == 0)
        def _():
            pltpu.sync_copy(shmax, red)
            m = red[0]
            for t in range(1, NS):
                m = jnp.maximum(m, red[t])
            mxb[...] = m
            pltpu.sync_copy(mxb, m2_h.at[c])

    return k(src, dst, asd_s, asd_d)


def _agg(src, dst, h3, e1, m2, N, K, C):
    E = src.shape[0]
    Wc = h3.shape[1]
    hpc = Wc // C
    Kp = K // 2
    B = 80
    PT = E // NS
    NB = PT // B
    RPT = N // NS
    NG = B // L
    zW = jnp.zeros((RPT, Wc), jnp.float32)
    zS = jnp.zeros((RPT, 16), jnp.float32)

    buf_types = [
        pltpu.VMEM((B,), jnp.int32),
        pltpu.VMEM((B,), jnp.int32),
        pltpu.VMEM((B,), jnp.int32),
        pltpu.VMEM((B,), jnp.int32),
        pltpu.VMEM((B, Wc), jnp.float32),
        pltpu.VMEM((B * 8 + 16,), jnp.float32),
        pltpu.SemaphoreType.DMA,
        pltpu.SemaphoreType.DMA,
        pltpu.SemaphoreType.DMA,
    ]

    @functools.partial(
        pl.kernel,
        out_type=[
            jax.ShapeDtypeStruct((K * N, Wc), jnp.float32),
            jax.ShapeDtypeStruct((N, 16), jnp.float32),
        ],
        mesh=_mesh(),
        scratch_types=buf_types + buf_types + [
            pltpu.VMEM((B, 16), jnp.float32),
            pltpu.VMEM((2, 16), jnp.float32),
            pltpu.VMEM_SHARED((N, Wc), jnp.float32),
            pltpu.VMEM_SHARED((N, 16), jnp.float32),
        ],
        compiler_params=_SC_PARAMS,
    )
    def k(src_h, dst_h, h3_h, e_h, m2_h, zW_h, zS_h, raw_h, s_h, *refs):
        buf0 = refs[0:9]
        buf1 = refs[9:18]
        exb, m2v, acc, sacc = refs[18:22]
        c = lax.axis_index("c")
        s = lax.axis_index("s")
        lane = lax.iota(jnp.int32, L)
        pltpu.sync_copy(m2_h, m2v)
        mv16 = jnp.maximum(m2v[0], m2v[1])
        l8 = lane & 7
        mpat = jnp.zeros((L,), jnp.float32)
        for hh in range(8):
            mpat = jnp.where(l8 == hh, mv16[hh], mpat)
        base = s * PT

        def off_of(t):
            return pl.multiple_of(base + t * B, 8)

        def issue_small(t, bf):
            off = off_of(t)
            pltpu.async_copy(src_h.at[pl.ds(off, B)], bf[0], bf[6])
            pltpu.async_copy(dst_h.at[pl.ds(off, B)], bf[1], bf[6])
            pltpu.async_copy(e_h.at[pl.ds(off * 8, B * 8)],
                             bf[5].at[pl.ds(0, B * 8)], bf[6])

        def wait_small(t, bf):
            off = off_of(t)
            pltpu.make_async_copy(src_h.at[pl.ds(off, B)], bf[0], bf[6]).wait()
            pltpu.make_async_copy(dst_h.at[pl.ds(off, B)], bf[1], bf[6]).wait()
            pltpu.make_async_copy(e_h.at[pl.ds(off * 8, B * 8)],
                                  bf[5].at[pl.ds(0, B * 8)], bf[6]).wait()

        def compute(bf, kk, chunk, mh_l):
            dstv, dscv, rows, ev = bf[1], bf[2], bf[4], bf[5]

            def cpi(i, _):
                dscv[pl.ds(i * L, L)] = dstv[pl.ds(i * L, L)]
                return 0

            lax.fori_loop(0, B // L, cpi, 0)

            def grp(g, _):
                rvec = g * L + lane
                exs_l = [
                    jnp.exp(plsc.load_gather(ev, [rvec * 8 + (chunk * hpc + hh)])
                            - mh_l[hh])
                    for hh in range(hpc)
                ]
                for r16 in range(L):
                    rsp = jnp.zeros((L,), jnp.int32) + (g * L + r16)
                    for hh in range(hpc):
                        exs = exs_l[hh][r16]
                        for j in range(C // L):
                            cv = lane + (hh * C + j * L)
                            v = plsc.load_gather(rows, [rsp, cv])
                            plsc.store_scatter(rows, [rsp, cv], v * exs)
                return 0

            lax.fori_loop(0, NG, grp, 0)

            if kk == 0:
                @pl.when(c == 0)
                def _():
                    def srow(r, _):
                        rsp = jnp.zeros((L,), jnp.int32) + r
                        v = ev[pl.ds(r * 8, 16)]
                        plsc.store_scatter(exb, [rsp, lane],
                                           jnp.where(lane < 8, jnp.exp(v - mpat), 0.0))
                        return 0

                    lax.fori_loop(0, B, srow, 0)
                    pltpu.sync_copy(exb, sacc.at[dscv], add=True)

        def mk_idx(t, bf, chunk):
            srcv, idxv = bf[0], bf[3]

            def idxb(i, _):
                idxv[pl.ds(i * L, L)] = srcv[pl.ds(i * L, L)] + chunk * N
                return 0

            lax.fori_loop(0, B // L, idxb, 0)

        def step(t, cur, nxt, kk, chunk, mh_l):
            wait_small(t + 1, nxt)
            mk_idx(t + 1, nxt, chunk)

            @pl.when(t >= 1)
            def _():
                pltpu.make_async_copy(nxt[4], acc.at[nxt[2]], nxt[8]).wait()

            pltpu.async_copy(h3_h.at[nxt[3]], nxt[4], nxt[7])
            pltpu.make_async_copy(h3_h.at[cur[3]], cur[4], cur[7]).wait()
            compute(cur, kk, chunk, mh_l)
            pltpu.async_copy(cur[4], acc.at[cur[2]], cur[8], add=True)

            @pl.when(t + 2 <= NB - 1)
            def _():
                issue_small(t + 2, cur)

        for kk in range(Kp):
            chunk = c * Kp + kk
            mh_l = [jnp.sum(jnp.where(lane == chunk * hpc + hh, mv16, 0.0))
                    for hh in range(hpc)]
            pltpu.sync_copy(zW_h, acc.at[pl.ds(s * RPT, RPT)])
            if kk == 0:
                @pl.when(c == 0)
                def _():
                    pltpu.sync_copy(zS_h, sacc.at[pl.ds(s * RPT, RPT)])
            plsc.subcore_barrier()

            issue_small(0, buf0)
            issue_small(1, buf1)
            wait_small(0, buf0)
            mk_idx(0, buf0, chunk)
            pltpu.async_copy(h3_h.at[buf0[3]], buf0[4], buf0[7])

            def pair(q, _):
                t = q * 2
                step(t, buf0, buf1, kk, chunk, mh_l)
                step(t + 1, buf1, buf0, kk, chunk, mh_l)
                return 0

            lax.fori_loop(0, (NB - 1) // 2, pair, 0)
            tl = NB - 1
            pltpu.make_async_copy(buf1[4], acc.at[buf1[2]], buf1[8]).wait()
            pltpu.make_async_copy(h3_h.at[buf0[3]], buf0[4], buf0[7]).wait()
            compute(buf0, kk, chunk, mh_l)
            pltpu.sync_copy(buf0[4], acc.at[buf0[2]], add=True)

            plsc.subcore_barrier()
            pltpu.sync_copy(acc.at[pl.ds(s * RPT, RPT)],
                            raw_h.at[pl.ds(chunk * N + s * RPT, RPT)])
            if kk == 0:
                @pl.when(c == 0)
                def _():
                    pltpu.sync_copy(sacc.at[pl.ds(s * RPT, RPT)],
                                    s_h.at[pl.ds(s * RPT, RPT)])
            plsc.subcore_barrier()

    return k(src, dst, h3, e1, m2, zW, zS)


def _att_w(att_s, att_d, F, C):
    hid = jnp.repeat(jnp.arange(8), C)
    onehot = (hid[:, None] == jnp.arange(8)[None, :]).astype(jnp.float32)
    AS = att_s.reshape(F, 1) * onehot
    AD = att_d.reshape(F, 1) * onehot
    z = jnp.zeros((F, 8), jnp.float32)
    return jnp.concatenate([AS, z, AD, z], axis=1)


def _layer(src, dst, xin, W, aw, mom, g, be, b, K, C, bn):
    N = xin.shape[0]
    h3, as_t, ad_t = _dense(xin, W, aw, mom, g, be, K=K, bn=bn)
    Wc = h3.shape[2]
    e1, m2 = _edge(src, dst, as_t, ad_t)
    raw, s = _agg(src, dst, h3.reshape(K * N, Wc), e1, m2, N, K, C)
    return _post(raw.reshape(K, N, Wc), s, b, K=K, C=C)


def kernel(x, edge_index, batch, W1, as1, ad1, b1, g1, be1, W2, as2, ad2, b2, g2,
           be2, W3, as3, ad3, b3, g3, be3, Wfc, bfc):
    N = x.shape[0]
    src = edge_index[0]
    dst = edge_index[1]
    xp = jnp.pad(x, ((0, 0), (0, 5)))
    W1p = jnp.pad(W1, ((0, 5), (0, 0)))
    aw1 = _att_w(as1, ad1, 128, 16)
    aw2 = _att_w(as2, ad2, 256, 32)
    aw3 = _att_w(as3, ad3, 512, 64)

    y1, mom1 = _layer(src, dst, xp, W1p, aw1, None, None, None, b1, K=2, C=16, bn=False)
    y2, mom2 = _layer(src, dst, y1, W2, aw2, mom1, g1, be1, b2, K=2, C=32, bn=True)
    y3, mom3 = _layer(src, dst, y2, W3, aw3, mom2, g2, be2, b3, K=4, C=64, bn=True)

    pooled = _pool(y3, batch.reshape(N, 1), mom3, g3, be3)
    return _fc(pooled, Wfc, bfc)


# speedup vs baseline: 25.9903x; 1.1223x over previous
import functools

import jax
import jax.numpy as jnp
from jax import lax
from jax.experimental import pallas as pl
from jax.experimental.pallas import tpu as pltpu
from jax.experimental.pallas import tpu_sc as plsc

NC, NS, L = 2, 16, 16


def _mesh():
    return plsc.VectorSubcoreMesh(
        core_axis_name="c", subcore_axis_name="s", num_cores=NC, num_subcores=NS
    )


_SC_PARAMS = pltpu.CompilerParams(
    use_tc_tiling_on_sc=False, needs_layout_passes=False
)


def _dense_body(x_ref, w_ref, aw_ref, *rest, bn, K, n):
    if bn:
        mom_ref, g_ref, be_ref, h_ref, as_ref, ad_ref = rest
        mu = mom_ref[0:1, :] / n
        var = mom_ref[1:2, :] / n - mu * mu
        xin = (x_ref[...] - mu) * lax.rsqrt(var + 1e-5) * g_ref[...] + be_ref[...]
        xin = jnp.maximum(xin, 0.0)
    else:
        h_ref, as_ref, ad_ref = rest
        xin = x_ref[...]
    h = jnp.dot(xin, w_ref[...], preferred_element_type=jnp.float32)
    Wc = h_ref.shape[2]
    for k in range(K):
        h_ref[k] = h[:, k * Wc:(k + 1) * Wc]
    asd = jnp.dot(h, aw_ref[...], preferred_element_type=jnp.float32)
    as_ref[...] = asd[:, 0:16]
    ad_ref[...] = asd[:, 16:32]


def _dense(xin, W, aw, mom, g, be, K, bn):
    N, Fin = xin.shape
    F = W.shape[1]
    Wc = F // K
    R = 1000
    body = functools.partial(_dense_body, bn=bn, K=K, n=N)
    in_specs = [
        pl.BlockSpec((R, Fin), lambda i: (i, 0)),
        pl.BlockSpec((Fin, F), lambda i: (0, 0)),
        pl.BlockSpec((F, 32), lambda i: (0, 0)),
    ]
    args = [xin, W, aw]
    if bn:
        in_specs += [
            pl.BlockSpec((2, Fin), lambda i: (0, 0)),
            pl.BlockSpec((1, Fin), lambda i: (0, 0)),
            pl.BlockSpec((1, Fin), lambda i: (0, 0)),
        ]
        args += [mom, g.reshape(1, Fin), be.reshape(1, Fin)]
    return pl.pallas_call(
        body,
        grid=(N // R,),
        in_specs=in_specs,
        out_specs=[
            pl.BlockSpec((K, R, Wc), lambda i: (0, i, 0)),
            pl.BlockSpec((R, 16), lambda i: (i, 0)),
            pl.BlockSpec((R, 16), lambda i: (i, 0)),
        ],
        out_shape=[
            jax.ShapeDtypeStruct((K, N, Wc), jnp.float32),
            jax.ShapeDtypeStruct((N, 16), jnp.float32),
            jax.ShapeDtypeStruct((N, 16), jnp.float32),
        ],
    )(*args)


def _post_body(raw_ref, s_ref, b_ref, y_ref, mom_ref, *, K, C):
    @pl.when(pl.program_id(0) == 0)
    def _():
        mom_ref[...] = jnp.zeros_like(mom_ref)

    raw = jnp.concatenate([raw_ref[k] for k in range(K)], axis=1)
    ssum = s_ref[0] + s_ref[1]
    sinv = 1.0 / (ssum[:, 0:8] + 1e-16)
    ys = [raw[:, hh * C:(hh + 1) * C] * sinv[:, hh:hh + 1] for hh in range(8)]
    y = jnp.concatenate(ys, axis=1) + b_ref[...]
    y_ref[...] = y
    mom_ref[0:1, :] += jnp.sum(y, axis=0, keepdims=True)
    mom_ref[1:2, :] += jnp.sum(y * y, axis=0, keepdims=True)


def _post(raw, s, b, K, C):
    _, N, Wc = raw.shape
    F = K * Wc
    R = 1000
    return pl.pallas_call(
        functools.partial(_post_body, K=K, C=C),
        grid=(N // R,),
        in_specs=[
            pl.BlockSpec((K, R, Wc), lambda i: (0, i, 0)),
            pl.BlockSpec((2, R, 16), lambda i: (0, i, 0)),
            pl.BlockSpec((1, F), lambda i: (0, 0)),
        ],
        out_specs=[
            pl.BlockSpec((R, F), lambda i: (i, 0)),
            pl.BlockSpec((2, F), lambda i: (0, 0)),
        ],
        out_shape=[
            jax.ShapeDtypeStruct((N, F), jnp.float32),
            jax.ShapeDtypeStruct((2, F), jnp.float32),
        ],
    )(raw, s.reshape(2, N, 16), b.reshape(1, F))


def _pool_body(y_ref, bid_ref, mom_ref, g_ref, be_ref, out_ref, *, n):
    @pl.when(pl.program_id(0) == 0)
    def _():
        out_ref[...] = jnp.full_like(out_ref[...], -jnp.inf)

    mu = mom_ref[0:1, :] / n
    var = mom_ref[1:2, :] / n - mu * mu
    y = (y_ref[...] - mu) * lax.rsqrt(var + 1e-5) * g_ref[...] + be_ref[...]
    y = jnp.maximum(y, 0.0)
    bid = bid_ref[...]
    lo = jnp.min(bid)
    hi = jnp.max(bid)

    def body(b, _):
        m = bid == b
        cm = jnp.max(jnp.where(m, y, -jnp.inf), axis=0, keepdims=True)
        out_ref[pl.ds(b, 1), :] = jnp.maximum(out_ref[pl.ds(b, 1), :], cm)
        return 0

    lax.fori_loop(lo, hi + 1, body, 0)


def _pool(y, bid, mom, g, be):
    N, F = y.shape
    R = 200
    return pl.pallas_call(
        functools.partial(_pool_body, n=N),
        grid=(N // R,),
        in_specs=[
            pl.BlockSpec((R, F), lambda i: (i, 0)),
            pl.BlockSpec((R, 1), lambda i: (i, 0)),
            pl.BlockSpec((2, F), lambda i: (0, 0)),
            pl.BlockSpec((1, F), lambda i: (0, 0)),
            pl.BlockSpec((1, F), lambda i: (0, 0)),
        ],
        out_specs=pl.BlockSpec((64, F), lambda i: (0, 0)),
        out_shape=jax.ShapeDtypeStruct((64, F), jnp.float32),
    )(y, bid, mom, g.reshape(1, F), be.reshape(1, F))


def _fc_body(p_ref, w_ref, b_ref, o_ref):
    p = p_ref[...]
    p = jnp.where(jnp.isfinite(p), p, 0.0)
    o_ref[...] = jnp.dot(p, w_ref[...], preferred_element_type=jnp.float32) + b_ref[...]


def _fc(pooled, Wfc, bfc):
    return pl.pallas_call(
        _fc_body,
        out_shape=jax.ShapeDtypeStruct((pooled.shape[0], Wfc.shape[1]), jnp.float32),
    )(pooled, Wfc, bfc.reshape(1, -1))


def _edge(src, dst, asd_s, asd_d):
    E = src.shape[0]
    B = 40
    PT = E // (NC * NS)
    NB = PT // B

    ebuf_types = [
        pltpu.VMEM((B,), jnp.int32),
        pltpu.VMEM((B,), jnp.int32),
        pltpu.VMEM((B, 16), jnp.float32),
        pltpu.VMEM((B, 16), jnp.float32),
        pltpu.VMEM((B * 8 + 16,), jnp.float32),
        pltpu.SemaphoreType.DMA,
        pltpu.SemaphoreType.DMA,
        pltpu.SemaphoreType.DMA,
    ]

    @functools.partial(
        pl.kernel,
        out_type=[
            jax.ShapeDtypeStruct((E * 8,), jnp.float32),
            jax.ShapeDtypeStruct((2, 16), jnp.float32),
        ],
        mesh=_mesh(),
        scratch_types=ebuf_types + ebuf_types + [
            pltpu.VMEM((NS, 16), jnp.float32),
            pltpu.VMEM((16,), jnp.float32),
            pltpu.VMEM_SHARED((NS, 16), jnp.float32),
        ],
        compiler_params=_SC_PARAMS,
    )
    def k(src_h, dst_h, as_h, ad_h, e_h, m2_h, *refs):
        buf0 = refs[0:8]
        buf1 = refs[8:16]
        red, mxb, shmax = refs[16:19]
        c = lax.axis_index("c")
        s = lax.axis_index("s")
        wid = c * NS + s
        lane = lax.iota(jnp.int32, L)
        neg = jnp.zeros((L,), jnp.float32) - jnp.inf
        base = wid * PT

        def off_of(t):
            return pl.multiple_of(base + t * B, 8)

        def issue_small(t, bf):
            off = off_of(t)
            pltpu.async_copy(src_h.at[pl.ds(off, B)], bf[0], bf[5])
            pltpu.async_copy(dst_h.at[pl.ds(off, B)], bf[1], bf[5])

        def wait_small(t, bf):
            off = off_of(t)
            pltpu.make_async_copy(src_h.at[pl.ds(off, B)], bf[0], bf[5]).wait()
            pltpu.make_async_copy(dst_h.at[pl.ds(off, B)], bf[1], bf[5]).wait()

        def issue_gather(bf):
            pltpu.async_copy(as_h.at[bf[0]], bf[2], bf[6])
            pltpu.async_copy(ad_h.at[bf[1]], bf[3], bf[6])

        def wait_gather(bf):
            pltpu.make_async_copy(as_h.at[bf[0]], bf[2], bf[6]).wait()
            pltpu.make_async_copy(ad_h.at[bf[1]], bf[3], bf[6]).wait()

        def compute(bf, mx):
            srows, drows, eblk = bf[2], bf[3], bf[4]
            for r in range(B):
                v = srows[r] + drows[r]
                e16 = jnp.maximum(v, 0.2 * v)
                eblk[pl.ds(r * 8, 16)] = e16
                mx = jnp.maximum(mx, jnp.where(lane < 8, e16, neg))
            return mx

        def estore(t, bf):
            pltpu.async_copy(bf[4].at[pl.ds(0, B * 8)],
                             e_h.at[pl.ds(off_of(t) * 8, B * 8)], bf[7])

        def wait_estore(t, bf):
            pltpu.make_async_copy(bf[4].at[pl.ds(0, B * 8)],
                                  e_h.at[pl.ds(off_of(t) * 8, B * 8)], bf[7]).wait()

        def step(t, cur, nxt, mx):
            wait_small(t + 1, nxt)

            @pl.when(t >= 1)
            def _():
                wait_estore(t - 1, nxt)

            issue_gather(nxt)
            wait_gather(cur)
            mx = compute(cur, mx)
            estore(t, cur)

            @pl.when(t + 2 <= NB - 1)
            def _():
                issue_small(t + 2, cur)

            return mx

        issue_small(0, buf0)
        issue_small(1, buf1)
        wait_small(0, buf0)
        issue_gather(buf0)

        def pair(q, mx):
            t = q * 2
            mx = step(t, buf0, buf1, mx)
            mx = step(t + 1, buf1, buf0, mx)
            return mx

        mx = lax.fori_loop(0, (NB - 1) // 2, pair, neg)
        wait_estore(NB - 2, buf1)
        wait_gather(buf0)
        mx = compute(buf0, mx)
        pltpu.sync_copy(buf0[4].at[pl.ds(0, B * 8)],
                        e_h.at[pl.ds(off_of(NB - 1) * 8, B * 8)])
        mxb[...] = mx
        pltpu.sync_copy(mxb, shmax.at[s])
        plsc.subcore_barrier()

        @pl.when(s == 0)
        def _():
            pltpu.sync_copy(shmax, red)
            m = red[0]
            for t in range(1, NS):
                m = jnp.maximum(m, red[t])
            mxb[...] = m
            pltpu.sync_copy(mxb, m2_h.at[c])

    return k(src, dst, asd_s, asd_d)


def _agg(src, dst, h3, e1, m2, N, K, C):
    E = src.shape[0]
    Wc = h3.shape[1]
    hpc = Wc // C
    Kp = K // 2
    B = 80
    PT = E // NS
    NB = PT // B
    RPT = N // NS
    NG = B // L
    zW = jnp.zeros((RPT, Wc), jnp.float32)
    zS = jnp.zeros((RPT, 16), jnp.float32)

    buf_types = [
        pltpu.VMEM((B,), jnp.int32),
        pltpu.VMEM((B,), jnp.int32),
        pltpu.VMEM((B,), jnp.int32),
        pltpu.VMEM((B,), jnp.int32),
        pltpu.VMEM((B, Wc), jnp.float32),
        pltpu.VMEM((B * 8 + 16,), jnp.float32),
        pltpu.SemaphoreType.DMA,
        pltpu.SemaphoreType.DMA,
        pltpu.SemaphoreType.DMA,
    ]

    @functools.partial(
        pl.kernel,
        out_type=[
            jax.ShapeDtypeStruct((K * N, Wc), jnp.float32),
            jax.ShapeDtypeStruct((2 * N, 16), jnp.float32),
        ],
        mesh=_mesh(),
        scratch_types=buf_types + buf_types + [
            pltpu.VMEM((B, 16), jnp.float32),
            pltpu.VMEM((2, 16), jnp.float32),
            pltpu.VMEM_SHARED((N, Wc), jnp.float32),
            pltpu.VMEM_SHARED((N, 16), jnp.float32),
        ],
        compiler_params=_SC_PARAMS,
    )
    def k(src_h, dst_h, h3_h, e_h, m2_h, zW_h, zS_h, raw_h, s_h, *refs):
        buf0 = refs[0:9]
        buf1 = refs[9:18]
        exb, m2v, acc, sacc = refs[18:22]
        c = lax.axis_index("c")
        s = lax.axis_index("s")
        lane = lax.iota(jnp.int32, L)
        pltpu.sync_copy(m2_h, m2v)
        mv16 = jnp.maximum(m2v[0], m2v[1])
        l8 = lane & 7
        mpat = jnp.zeros((L,), jnp.float32)
        for hh in range(8):
            mpat = jnp.where(l8 == hh, mv16[hh], mpat)
        base = s * PT

        def off_of(t):
            return pl.multiple_of(base + t * B, 8)

        def issue_small(t, bf):
            off = off_of(t)
            pltpu.async_copy(src_h.at[pl.ds(off, B)], bf[0], bf[6])
            pltpu.async_copy(dst_h.at[pl.ds(off, B)], bf[1], bf[6])
            pltpu.async_copy(e_h.at[pl.ds(off * 8, B * 8)],
                             bf[5].at[pl.ds(0, B * 8)], bf[6])

        def wait_small(t, bf):
            off = off_of(t)
            pltpu.make_async_copy(src_h.at[pl.ds(off, B)], bf[0], bf[6]).wait()
            pltpu.make_async_copy(dst_h.at[pl.ds(off, B)], bf[1], bf[6]).wait()
            pltpu.make_async_copy(e_h.at[pl.ds(off * 8, B * 8)],
                                  bf[5].at[pl.ds(0, B * 8)], bf[6]).wait()

        def compute(t, bf, kk, chunk, mh_l):
            dstv, dscv, rows, ev = bf[1], bf[2], bf[4], bf[5]

            def cpi(i, _):
                dscv[pl.ds(i * L, L)] = dstv[pl.ds(i * L, L)]
                return 0

            lax.fori_loop(0, B // L, cpi, 0)

            def grp(g, _):
                rvec = g * L + lane
                exs_l = [
                    jnp.exp(plsc.load_gather(ev, [rvec * 8 + (chunk * hpc + hh)])
                            - mh_l[hh])
                    for hh in range(hpc)
                ]
                for r16 in range(L):
                    rsp = jnp.zeros((L,), jnp.int32) + (g * L + r16)
                    for hh in range(hpc):
                        exs = exs_l[hh][r16]
                        for j in range(C // L):
                            cv = lane + (hh * C + j * L)
                            v = plsc.load_gather(rows, [rsp, cv])
                            plsc.store_scatter(rows, [rsp, cv], v * exs)
                return 0

            lax.fori_loop(0, NG, grp, 0)

            if kk == 0:
                @pl.when(jnp.where(c == 0, t < NB // 2, t >= NB // 2))
                def _():
                    def srow(r, _):
                        rsp = jnp.zeros((L,), jnp.int32) + r
                        v = ev[pl.ds(r * 8, 16)]
                        plsc.store_scatter(exb, [rsp, lane],
                                           jnp.where(lane < 8, jnp.exp(v - mpat), 0.0))
                        return 0

                    lax.fori_loop(0, B, srow, 0)
                    pltpu.sync_copy(exb, sacc.at[dscv], add=True)

        def mk_idx(t, bf, chunk):
            srcv, idxv = bf[0], bf[3]

            def idxb(i, _):
                idxv[pl.ds(i * L, L)] = srcv[pl.ds(i * L, L)] + chunk * N
                return 0

            lax.fori_loop(0, B // L, idxb, 0)

        def step(t, cur, nxt, kk, chunk, mh_l):
            wait_small(t + 1, nxt)
            mk_idx(t + 1, nxt, chunk)

            @pl.when(t >= 1)
            def _():
                pltpu.make_async_copy(nxt[4], acc.at[nxt[2]], nxt[8]).wait()

            pltpu.async_copy(h3_h.at[nxt[3]], nxt[4], nxt[7])
            pltpu.make_async_copy(h3_h.at[cur[3]], cur[4], cur[7]).wait()
            compute(t, cur, kk, chunk, mh_l)
            pltpu.async_copy(cur[4], acc.at[cur[2]], cur[8], add=True)

            @pl.when(t + 2 <= NB - 1)
            def _():
                issue_small(t + 2, cur)

        for kk in range(Kp):
            chunk = c * Kp + kk
            mh_l = [jnp.sum(jnp.where(lane == chunk * hpc + hh, mv16, 0.0))
                    for hh in range(hpc)]
            pltpu.sync_copy(zW_h, acc.at[pl.ds(s * RPT, RPT)])
            if kk == 0:
                pltpu.sync_copy(zS_h, sacc.at[pl.ds(s * RPT, RPT)])
            plsc.subcore_barrier()

            issue_small(0, buf0)
            issue_small(1, buf1)
            wait_small(0, buf0)
            mk_idx(0, buf0, chunk)
            pltpu.async_copy(h3_h.at[buf0[3]], buf0[4], buf0[7])

            def pair(q, _):
                t = q * 2
                step(t, buf0, buf1, kk, chunk, mh_l)
                step(t + 1, buf1, buf0, kk, chunk, mh_l)
                return 0

            lax.fori_loop(0, (NB - 1) // 2, pair, 0)
            tl = NB - 1
            pltpu.make_async_copy(buf1[4], acc.at[buf1[2]], buf1[8]).wait()
            pltpu.make_async_copy(h3_h.at[buf0[3]], buf0[4], buf0[7]).wait()
            compute(jnp.int32(tl), buf0, kk, chunk, mh_l)
            pltpu.sync_copy(buf0[4], acc.at[buf0[2]], add=True)

            plsc.subcore_barrier()
            pltpu.sync_copy(acc.at[pl.ds(s * RPT, RPT)],
                            raw_h.at[pl.ds(chunk * N + s * RPT, RPT)])
            if kk == 0:
                pltpu.sync_copy(sacc.at[pl.ds(s * RPT, RPT)],
                                s_h.at[pl.ds(c * N + s * RPT, RPT)])
            plsc.subcore_barrier()

    return k(src, dst, h3, e1, m2, zW, zS)


def _att_w(att_s, att_d, F, C):
    hid = jnp.repeat(jnp.arange(8), C)
    onehot = (hid[:, None] == jnp.arange(8)[None, :]).astype(jnp.float32)
    AS = att_s.reshape(F, 1) * onehot
    AD = att_d.reshape(F, 1) * onehot
    z = jnp.zeros((F, 8), jnp.float32)
    return jnp.concatenate([AS, z, AD, z], axis=1)


def _layer(src, dst, xin, W, aw, mom, g, be, b, K, C, bn):
    N = xin.shape[0]
    h3, as_t, ad_t = _dense(xin, W, aw, mom, g, be, K=K, bn=bn)
    Wc = h3.shape[2]
    e1, m2 = _edge(src, dst, as_t, ad_t)
    raw, s = _agg(src, dst, h3.reshape(K * N, Wc), e1, m2, N, K, C)
    return _post(raw.reshape(K, N, Wc), s, b, K=K, C=C)


def kernel(x, edge_index, batch, W1, as1, ad1, b1, g1, be1, W2, as2, ad2, b2, g2,
           be2, W3, as3, ad3, b3, g3, be3, Wfc, bfc):
    N = x.shape[0]
    src = edge_index[0]
    dst = edge_index[1]
    xp = jnp.pad(x, ((0, 0), (0, 5)))
    W1p = jnp.pad(W1, ((0, 5), (0, 0)))
    aw1 = _att_w(as1, ad1, 128, 16)
    aw2 = _att_w(as2, ad2, 256, 32)
    aw3 = _att_w(as3, ad3, 512, 64)

    y1, mom1 = _layer(src, dst, xp, W1p, aw1, None, None, None, b1, K=2, C=16, bn=False)
    y2, mom2 = _layer(src, dst, y1, W2, aw2, mom1, g1, be1, b2, K=2, C=32, bn=True)
    y3, mom3 = _layer(src, dst, y2, W3, aw3, mom2, g2, be2, b3, K=4, C=64, bn=True)

    pooled = _pool(y3, batch.reshape(N, 1), mom3, g3, be3)
    return _fc(pooled, Wfc, bfc)


# speedup vs baseline: 26.4410x; 1.0173x over previous
import functools

import jax
import jax.numpy as jnp
from jax import lax
from jax.experimental import pallas as pl
from jax.experimental.pallas import tpu as pltpu
from jax.experimental.pallas import tpu_sc as plsc

NC, NS, L = 2, 16, 16


def _mesh():
    return plsc.VectorSubcoreMesh(
        core_axis_name="c", subcore_axis_name="s", num_cores=NC, num_subcores=NS
    )


_SC_PARAMS = pltpu.CompilerParams(
    use_tc_tiling_on_sc=False, needs_layout_passes=False
)


def _dense_body(x_ref, w_ref, aw_ref, *rest, bn, K, n):
    if bn:
        mom_ref, g_ref, be_ref, h_ref, as_ref, ad_ref = rest
        mu = mom_ref[0:1, :] / n
        var = mom_ref[1:2, :] / n - mu * mu
        xin = (x_ref[...] - mu) * lax.rsqrt(var + 1e-5) * g_ref[...] + be_ref[...]
        xin = jnp.maximum(xin, 0.0)
    else:
        h_ref, as_ref, ad_ref = rest
        xin = x_ref[...]
    h = jnp.dot(xin, w_ref[...], preferred_element_type=jnp.float32)
    Wc = h_ref.shape[2]
    for k in range(K):
        h_ref[k] = h[:, k * Wc:(k + 1) * Wc]
    asd = jnp.dot(h, aw_ref[...], preferred_element_type=jnp.float32)
    as_ref[...] = asd[:, 0:16]
    ad_ref[...] = asd[:, 16:32]


def _dense(xin, W, aw, mom, g, be, K, bn):
    N, Fin = xin.shape
    F = W.shape[1]
    Wc = F // K
    R = 1000
    body = functools.partial(_dense_body, bn=bn, K=K, n=N)
    in_specs = [
        pl.BlockSpec((R, Fin), lambda i: (i, 0)),
        pl.BlockSpec((Fin, F), lambda i: (0, 0)),
        pl.BlockSpec((F, 32), lambda i: (0, 0)),
    ]
    args = [xin, W, aw]
    if bn:
        in_specs += [
            pl.BlockSpec((2, Fin), lambda i: (0, 0)),
            pl.BlockSpec((1, Fin), lambda i: (0, 0)),
            pl.BlockSpec((1, Fin), lambda i: (0, 0)),
        ]
        args += [mom, g.reshape(1, Fin), be.reshape(1, Fin)]
    return pl.pallas_call(
        body,
        grid=(N // R,),
        in_specs=in_specs,
        out_specs=[
            pl.BlockSpec((K, R, Wc), lambda i: (0, i, 0)),
            pl.BlockSpec((R, 16), lambda i: (i, 0)),
            pl.BlockSpec((R, 16), lambda i: (i, 0)),
        ],
        out_shape=[
            jax.ShapeDtypeStruct((K, N, Wc), jnp.float32),
            jax.ShapeDtypeStruct((N, 16), jnp.float32),
            jax.ShapeDtypeStruct((N, 16), jnp.float32),
        ],
    )(*args)


def _post_body(raw_ref, s_ref, b_ref, y_ref, mom_ref, *, K, C):
    @pl.when(pl.program_id(0) == 0)
    def _():
        mom_ref[...] = jnp.zeros_like(mom_ref)

    raw = jnp.concatenate([raw_ref[k] for k in range(K)], axis=1)
    ssum = s_ref[0] + s_ref[1]
    sinv = 1.0 / (ssum[:, 0:8] + 1e-16)
    ys = [raw[:, hh * C:(hh + 1) * C] * sinv[:, hh:hh + 1] for hh in range(8)]
    y = jnp.concatenate(ys, axis=1) + b_ref[...]
    y_ref[...] = y
    mom_ref[0:1, :] += jnp.sum(y, axis=0, keepdims=True)
    mom_ref[1:2, :] += jnp.sum(y * y, axis=0, keepdims=True)


def _post(raw, s, b, K, C):
    _, N, Wc = raw.shape
    F = K * Wc
    R = 1000
    return pl.pallas_call(
        functools.partial(_post_body, K=K, C=C),
        grid=(N // R,),
        in_specs=[
            pl.BlockSpec((K, R, Wc), lambda i: (0, i, 0)),
            pl.BlockSpec((2, R, 16), lambda i: (0, i, 0)),
            pl.BlockSpec((1, F), lambda i: (0, 0)),
        ],
        out_specs=[
            pl.BlockSpec((R, F), lambda i: (i, 0)),
            pl.BlockSpec((2, F), lambda i: (0, 0)),
        ],
        out_shape=[
            jax.ShapeDtypeStruct((N, F), jnp.float32),
            jax.ShapeDtypeStruct((2, F), jnp.float32),
        ],
    )(raw, s.reshape(2, N, 16), b.reshape(1, F))


def _pool_body(y_ref, bid_ref, mom_ref, g_ref, be_ref, out_ref, *, n):
    @pl.when(pl.program_id(0) == 0)
    def _():
        out_ref[...] = jnp.full_like(out_ref[...], -jnp.inf)

    mu = mom_ref[0:1, :] / n
    var = mom_ref[1:2, :] / n - mu * mu
    y = (y_ref[...] - mu) * lax.rsqrt(var + 1e-5) * g_ref[...] + be_ref[...]
    y = jnp.maximum(y, 0.0)
    bid = bid_ref[...]
    lo = jnp.min(bid)
    hi = jnp.max(bid)

    def body(b, _):
        m = bid == b
        cm = jnp.max(jnp.where(m, y, -jnp.inf), axis=0, keepdims=True)
        out_ref[pl.ds(b, 1), :] = jnp.maximum(out_ref[pl.ds(b, 1), :], cm)
        return 0

    lax.fori_loop(lo, hi + 1, body, 0)


def _pool(y, bid, mom, g, be):
    N, F = y.shape
    R = 200
    return pl.pallas_call(
        functools.partial(_pool_body, n=N),
        grid=(N // R,),
        in_specs=[
            pl.BlockSpec((R, F), lambda i: (i, 0)),
            pl.BlockSpec((R, 1), lambda i: (i, 0)),
            pl.BlockSpec((2, F), lambda i: (0, 0)),
            pl.BlockSpec((1, F), lambda i: (0, 0)),
            pl.BlockSpec((1, F), lambda i: (0, 0)),
        ],
        out_specs=pl.BlockSpec((64, F), lambda i: (0, 0)),
        out_shape=jax.ShapeDtypeStruct((64, F), jnp.float32),
    )(y, bid, mom, g.reshape(1, F), be.reshape(1, F))


def _fc_body(p_ref, w_ref, b_ref, o_ref):
    p = p_ref[...]
    p = jnp.where(jnp.isfinite(p), p, 0.0)
    o_ref[...] = jnp.dot(p, w_ref[...], preferred_element_type=jnp.float32) + b_ref[...]


def _fc(pooled, Wfc, bfc):
    return pl.pallas_call(
        _fc_body,
        out_shape=jax.ShapeDtypeStruct((pooled.shape[0], Wfc.shape[1]), jnp.float32),
    )(pooled, Wfc, bfc.reshape(1, -1))


def _edge(src, dst, asd_s, asd_d):
    E = src.shape[0]
    B = 40
    PT = E // (NC * NS)
    NB = PT // B

    ebuf_types = [
        pltpu.VMEM((B,), jnp.int32),
        pltpu.VMEM((B,), jnp.int32),
        pltpu.VMEM((B, 16), jnp.float32),
        pltpu.VMEM((B, 16), jnp.float32),
        pltpu.VMEM((B * 8 + 16,), jnp.float32),
        pltpu.SemaphoreType.DMA,
        pltpu.SemaphoreType.DMA,
        pltpu.SemaphoreType.DMA,
    ]

    @functools.partial(
        pl.kernel,
        out_type=[
            jax.ShapeDtypeStruct((E * 8,), jnp.float32),
            jax.ShapeDtypeStruct((2, 16), jnp.float32),
        ],
        mesh=_mesh(),
        scratch_types=ebuf_types + ebuf_types + [
            pltpu.VMEM((NS, 16), jnp.float32),
            pltpu.VMEM((16,), jnp.float32),
            pltpu.VMEM_SHARED((NS, 16), jnp.float32),
        ],
        compiler_params=_SC_PARAMS,
    )
    def k(src_h, dst_h, as_h, ad_h, e_h, m2_h, *refs):
        buf0 = refs[0:8]
        buf1 = refs[8:16]
        red, mxb, shmax = refs[16:19]
        c = lax.axis_index("c")
        s = lax.axis_index("s")
        wid = c * NS + s
        lane = lax.iota(jnp.int32, L)
        neg = jnp.zeros((L,), jnp.float32) - jnp.inf
        base = wid * PT

        def off_of(t):
            return pl.multiple_of(base + t * B, 8)

        def issue_small(t, bf):
            off = off_of(t)
            pltpu.async_copy(src_h.at[pl.ds(off, B)], bf[0], bf[5])
            pltpu.async_copy(dst_h.at[pl.ds(off, B)], bf[1], bf[5])

        def wait_small(t, bf):
            off = off_of(t)
            pltpu.make_async_copy(src_h.at[pl.ds(off, B)], bf[0], bf[5]).wait()
            pltpu.make_async_copy(dst_h.at[pl.ds(off, B)], bf[1], bf[5]).wait()

        def issue_gather(bf):
            pltpu.async_copy(as_h.at[bf[0]], bf[2], bf[6])
            pltpu.async_copy(ad_h.at[bf[1]], bf[3], bf[6])

        def wait_gather(bf):
            pltpu.make_async_copy(as_h.at[bf[0]], bf[2], bf[6]).wait()
            pltpu.make_async_copy(ad_h.at[bf[1]], bf[3], bf[6]).wait()

        def compute(bf, mx):
            srows, drows, eblk = bf[2], bf[3], bf[4]
            for r in range(B):
                v = srows[r] + drows[r]
                e16 = jnp.maximum(v, 0.2 * v)
                eblk[pl.ds(r * 8, 16)] = e16
                mx = jnp.maximum(mx, jnp.where(lane < 8, e16, neg))
            return mx

        def estore(t, bf):
            pltpu.async_copy(bf[4].at[pl.ds(0, B * 8)],
                             e_h.at[pl.ds(off_of(t) * 8, B * 8)], bf[7])

        def wait_estore(t, bf):
            pltpu.make_async_copy(bf[4].at[pl.ds(0, B * 8)],
                                  e_h.at[pl.ds(off_of(t) * 8, B * 8)], bf[7]).wait()

        def step(t, cur, nxt, mx):
            wait_small(t + 1, nxt)

            @pl.when(t >= 1)
            def _():
                wait_estore(t - 1, nxt)

            issue_gather(nxt)
            wait_gather(cur)
            mx = compute(cur, mx)
            estore(t, cur)

            @pl.when(t + 2 <= NB - 1)
            def _():
                issue_small(t + 2, cur)

            return mx

        issue_small(0, buf0)
        issue_small(1, buf1)
        wait_small(0, buf0)
        issue_gather(buf0)

        def pair(q, mx):
            t = q * 2
            mx = step(t, buf0, buf1, mx)
            mx = step(t + 1, buf1, buf0, mx)
            return mx

        mx = lax.fori_loop(0, (NB - 1) // 2, pair, neg)
        wait_estore(NB - 2, buf1)
        wait_gather(buf0)
        mx = compute(buf0, mx)
        pltpu.sync_copy(buf0[4].at[pl.ds(0, B * 8)],
                        e_h.at[pl.ds(off_of(NB - 1) * 8, B * 8)])
        mxb[...] = mx
        pltpu.sync_copy(mxb, shmax.at[s])
        plsc.subcore_barrier()

        @pl.when(s == 0)
        def _():
            pltpu.sync_copy(shmax, red)
            m = red[0]
            for t in range(1, NS):
                m = jnp.maximum(m, red[t])
            mxb[...] = m
            pltpu.sync_copy(mxb, m2_h.at[c])

    return k(src, dst, asd_s, asd_d)


def _agg(src, dst, h3, e1, m2, N, K, C):
    E = src.shape[0]
    Wc = h3.shape[1]
    hpc = Wc // C
    Kp = K // 2
    B = 128
    NB = (E // B) // NS
    XTRA = (E // B) % NS
    RPT = N // NS
    NG = B // L
    zW = jnp.zeros((RPT, Wc), jnp.float32)
    zS = jnp.zeros((RPT, 16), jnp.float32)

    buf_types = [
        pltpu.VMEM((B,), jnp.int32),
        pltpu.VMEM((B,), jnp.int32),
        pltpu.VMEM((B,), jnp.int32),
        pltpu.VMEM((B,), jnp.int32),
        pltpu.VMEM((B, Wc), jnp.float32),
        pltpu.VMEM((B * 8 + 16,), jnp.float32),
        pltpu.SemaphoreType.DMA,
        pltpu.SemaphoreType.DMA,
        pltpu.SemaphoreType.DMA,
    ]

    @functools.partial(
        pl.kernel,
        out_type=[
            jax.ShapeDtypeStruct((K * N, Wc), jnp.float32),
            jax.ShapeDtypeStruct((2 * N, 16), jnp.float32),
        ],
        mesh=_mesh(),
        scratch_types=buf_types + buf_types + [
            pltpu.VMEM((B, 16), jnp.float32),
            pltpu.VMEM((2, 16), jnp.float32),
            pltpu.VMEM_SHARED((N, Wc), jnp.float32),
            pltpu.VMEM_SHARED((N, 16), jnp.float32),
        ],
        compiler_params=_SC_PARAMS,
    )
    def k(src_h, dst_h, h3_h, e_h, m2_h, zW_h, zS_h, raw_h, s_h, *refs):
        buf0 = refs[0:9]
        buf1 = refs[9:18]
        exb, m2v, acc, sacc = refs[18:22]
        c = lax.axis_index("c")
        s = lax.axis_index("s")
        lane = lax.iota(jnp.int32, L)
        pltpu.sync_copy(m2_h, m2v)
        mv16 = jnp.maximum(m2v[0], m2v[1])
        l8 = lane & 7
        mpat = jnp.zeros((L,), jnp.float32)
        for hh in range(8):
            mpat = jnp.where(l8 == hh, mv16[hh], mpat)
        base = s * NB * B

        def off_of(t):
            return pl.multiple_of(base + t * B, 8)

        def issue_small(t, bf):
            off = off_of(t)
            pltpu.async_copy(src_h.at[pl.ds(off, B)], bf[0], bf[6])
            pltpu.async_copy(dst_h.at[pl.ds(off, B)], bf[1], bf[6])
            pltpu.async_copy(e_h.at[pl.ds(off * 8, B * 8)],
                             bf[5].at[pl.ds(0, B * 8)], bf[6])

        def wait_small(t, bf):
            off = off_of(t)
            pltpu.make_async_copy(src_h.at[pl.ds(off, B)], bf[0], bf[6]).wait()
            pltpu.make_async_copy(dst_h.at[pl.ds(off, B)], bf[1], bf[6]).wait()
            pltpu.make_async_copy(e_h.at[pl.ds(off * 8, B * 8)],
                                  bf[5].at[pl.ds(0, B * 8)], bf[6]).wait()

        def compute(t, bf, kk, chunk, mh_l):
            dstv, dscv, rows, ev = bf[1], bf[2], bf[4], bf[5]

            def cpi(i, _):
                dscv[pl.ds(i * L, L)] = dstv[pl.ds(i * L, L)]
                return 0

            lax.fori_loop(0, B // L, cpi, 0)

            def grp(g, _):
                rvec = g * L + lane
                exs_l = [
                    jnp.exp(plsc.load_gather(ev, [rvec * 8 + (chunk * hpc + hh)])
                            - mh_l[hh])
                    for hh in range(hpc)
                ]
                for r16 in range(L):
                    rsp = jnp.zeros((L,), jnp.int32) + (g * L + r16)
                    for hh in range(hpc):
                        exs = exs_l[hh][r16]
                        for j in range(C // L):
                            cv = lane + (hh * C + j * L)
                            v = plsc.load_gather(rows, [rsp, cv])
                            plsc.store_scatter(rows, [rsp, cv], v * exs)
                return 0

            lax.fori_loop(0, NG, grp, 0)

            if kk == 0:
                @pl.when(jnp.where(c == 0, t < NB // 2, t >= NB // 2))
                def _():
                    def srow(r, _):
                        rsp = jnp.zeros((L,), jnp.int32) + r
                        v = ev[pl.ds(r * 8, 16)]
                        plsc.store_scatter(exb, [rsp, lane],
                                           jnp.where(lane < 8, jnp.exp(v - mpat), 0.0))
                        return 0

                    lax.fori_loop(0, B, srow, 0)
                    pltpu.sync_copy(exb, sacc.at[dscv], add=True)

        def mk_idx(t, bf, chunk):
            srcv, idxv = bf[0], bf[3]

            def idxb(i, _):
                idxv[pl.ds(i * L, L)] = srcv[pl.ds(i * L, L)] + chunk * N
                return 0

            lax.fori_loop(0, B // L, idxb, 0)

        def step(t, cur, nxt, kk, chunk, mh_l):
            wait_small(t + 1, nxt)
            mk_idx(t + 1, nxt, chunk)

            @pl.when(t >= 1)
            def _():
                pltpu.make_async_copy(nxt[4], acc.at[nxt[2]], nxt[8]).wait()

            pltpu.async_copy(h3_h.at[nxt[3]], nxt[4], nxt[7])
            pltpu.make_async_copy(h3_h.at[cur[3]], cur[4], cur[7]).wait()
            compute(t, cur, kk, chunk, mh_l)
            pltpu.async_copy(cur[4], acc.at[cur[2]], cur[8], add=True)

            @pl.when(t + 2 <= NB - 1)
            def _():
                issue_small(t + 2, cur)

        for kk in range(Kp):
            chunk = c * Kp + kk
            mh_l = [jnp.sum(jnp.where(lane == chunk * hpc + hh, mv16, 0.0))
                    for hh in range(hpc)]
            pltpu.sync_copy(zW_h, acc.at[pl.ds(s * RPT, RPT)])
            if kk == 0:
                pltpu.sync_copy(zS_h, sacc.at[pl.ds(s * RPT, RPT)])
            plsc.subcore_barrier()

            issue_small(0, buf0)
            issue_small(1, buf1)
            wait_small(0, buf0)
            mk_idx(0, buf0, chunk)
            pltpu.async_copy(h3_h.at[buf0[3]], buf0[4], buf0[7])

            def pair(q, _):
                t = q * 2
                step(t, buf0, buf1, kk, chunk, mh_l)
                step(t + 1, buf1, buf0, kk, chunk, mh_l)
                return 0

            lax.fori_loop(0, (NB - 2) // 2, pair, 0)
            step(jnp.int32(NB - 2), buf0, buf1, kk, chunk, mh_l)
            pltpu.make_async_copy(buf0[4], acc.at[buf0[2]], buf0[8]).wait()
            pltpu.make_async_copy(h3_h.at[buf1[3]], buf1[4], buf1[7]).wait()
            compute(jnp.int32(NB - 1), buf1, kk, chunk, mh_l)
            pltpu.sync_copy(buf1[4], acc.at[buf1[2]], add=True)

            @pl.when(s < XTRA)
            def _():
                offx = pl.multiple_of((NS * NB + s) * B, 8)
                pltpu.sync_copy(src_h.at[pl.ds(offx, B)], buf0[0])
                pltpu.sync_copy(dst_h.at[pl.ds(offx, B)], buf0[1])
                pltpu.sync_copy(e_h.at[pl.ds(offx * 8, B * 8)],
                                buf0[5].at[pl.ds(0, B * 8)])
                mk_idx(0, buf0, chunk)
                pltpu.async_copy(h3_h.at[buf0[3]], buf0[4], buf0[7]).wait()
                compute(jnp.int32(0), buf0, kk, chunk, mh_l)
                pltpu.sync_copy(buf0[4], acc.at[buf0[2]], add=True)

            plsc.subcore_barrier()
            pltpu.sync_copy(acc.at[pl.ds(s * RPT, RPT)],
                            raw_h.at[pl.ds(chunk * N + s * RPT, RPT)])
            if kk == 0:
                pltpu.sync_copy(sacc.at[pl.ds(s * RPT, RPT)],
                                s_h.at[pl.ds(c * N + s * RPT, RPT)])
            plsc.subcore_barrier()

    return k(src, dst, h3, e1, m2, zW, zS)


def _att_w(att_s, att_d, F, C):
    hid = jnp.repeat(jnp.arange(8), C)
    onehot = (hid[:, None] == jnp.arange(8)[None, :]).astype(jnp.float32)
    AS = att_s.reshape(F, 1) * onehot
    AD = att_d.reshape(F, 1) * onehot
    z = jnp.zeros((F, 8), jnp.float32)
    return jnp.concatenate([AS, z, AD, z], axis=1)


def _layer(src, dst, xin, W, aw, mom, g, be, b, K, C, bn):
    N = xin.shape[0]
    h3, as_t, ad_t = _dense(xin, W, aw, mom, g, be, K=K, bn=bn)
    Wc = h3.shape[2]
    e1, m2 = _edge(src, dst, as_t, ad_t)
    raw, s = _agg(src, dst, h3.reshape(K * N, Wc), e1, m2, N, K, C)
    return _post(raw.reshape(K, N, Wc), s, b, K=K, C=C)


def kernel(x, edge_index, batch, W1, as1, ad1, b1, g1, be1, W2, as2, ad2, b2, g2,
           be2, W3, as3, ad3, b3, g3, be3, Wfc, bfc):
    N = x.shape[0]
    src = edge_index[0]
    dst = edge_index[1]
    xp = jnp.pad(x, ((0, 0), (0, 5)))
    W1p = jnp.pad(W1, ((0, 5), (0, 0)))
    aw1 = _att_w(as1, ad1, 128, 16)
    aw2 = _att_w(as2, ad2, 256, 32)
    aw3 = _att_w(as3, ad3, 512, 64)

    y1, mom1 = _layer(src, dst, xp, W1p, aw1, None, None, None, b1, K=2, C=16, bn=False)
    y2, mom2 = _layer(src, dst, y1, W2, aw2, mom1, g1, be1, b2, K=2, C=32, bn=True)
    y3, mom3 = _layer(src, dst, y2, W3, aw3, mom2, g2, be2, b3, K=4, C=64, bn=True)

    pooled = _pool(y3, batch.reshape(N, 1), mom3, g3, be3)
    return _fc(pooled, Wfc, bfc)


# speedup vs baseline: 27.7317x; 1.0488x over previous
import functools

import jax
import jax.numpy as jnp
from jax import lax
from jax.experimental import pallas as pl
from jax.experimental.pallas import tpu as pltpu
from jax.experimental.pallas import tpu_sc as plsc

NC, NS, L = 2, 16, 16


def _mesh():
    return plsc.VectorSubcoreMesh(
        core_axis_name="c", subcore_axis_name="s", num_cores=NC, num_subcores=NS
    )


_SC_PARAMS = pltpu.CompilerParams(
    use_tc_tiling_on_sc=False, needs_layout_passes=False
)


def _dense_body(x_ref, w_ref, aw_ref, *rest, bn, K, n):
    if bn:
        mom_ref, g_ref, be_ref, h_ref, as_ref, ad_ref = rest
        mu = mom_ref[0:1, :] / n
        var = mom_ref[1:2, :] / n - mu * mu
        xin = (x_ref[...] - mu) * lax.rsqrt(var + 1e-5) * g_ref[...] + be_ref[...]
        xin = jnp.maximum(xin, 0.0)
    else:
        h_ref, as_ref, ad_ref = rest
        xin = x_ref[...]
    h = jnp.dot(xin, w_ref[...], preferred_element_type=jnp.float32)
    Wc = h_ref.shape[2]
    for k in range(K):
        h_ref[k] = h[:, k * Wc:(k + 1) * Wc]
    asd = jnp.dot(h, aw_ref[...], preferred_element_type=jnp.float32)
    as_ref[...] = asd[:, 0:16]
    ad_ref[...] = asd[:, 16:32]


def _dense(xin, W, aw, mom, g, be, K, bn):
    N, Fin = xin.shape
    F = W.shape[1]
    Wc = F // K
    R = 1000
    body = functools.partial(_dense_body, bn=bn, K=K, n=N)
    in_specs = [
        pl.BlockSpec((R, Fin), lambda i: (i, 0)),
        pl.BlockSpec((Fin, F), lambda i: (0, 0)),
        pl.BlockSpec((F, 32), lambda i: (0, 0)),
    ]
    args = [xin, W, aw]
    if bn:
        in_specs += [
            pl.BlockSpec((2, Fin), lambda i: (0, 0)),
            pl.BlockSpec((1, Fin), lambda i: (0, 0)),
            pl.BlockSpec((1, Fin), lambda i: (0, 0)),
        ]
        args += [mom, g.reshape(1, Fin), be.reshape(1, Fin)]
    return pl.pallas_call(
        body,
        grid=(N // R,),
        in_specs=in_specs,
        out_specs=[
            pl.BlockSpec((K, R, Wc), lambda i: (0, i, 0)),
            pl.BlockSpec((R, 16), lambda i: (i, 0)),
            pl.BlockSpec((R, 16), lambda i: (i, 0)),
        ],
        out_shape=[
            jax.ShapeDtypeStruct((K, N, Wc), jnp.float32),
            jax.ShapeDtypeStruct((N, 16), jnp.float32),
            jax.ShapeDtypeStruct((N, 16), jnp.float32),
        ],
    )(*args)


def _post_body(raw_ref, s_ref, b_ref, y_ref, mom_ref, *, K, C):
    @pl.when(pl.program_id(0) == 0)
    def _():
        mom_ref[...] = jnp.zeros_like(mom_ref)

    raw = jnp.concatenate([raw_ref[k] for k in range(K)], axis=1)
    ssum = s_ref[0] + s_ref[1]
    sinv = 1.0 / (ssum[:, 0:8] + 1e-16)
    ys = [raw[:, hh * C:(hh + 1) * C] * sinv[:, hh:hh + 1] for hh in range(8)]
    y = jnp.concatenate(ys, axis=1) + b_ref[...]
    y_ref[...] = y
    mom_ref[0:1, :] += jnp.sum(y, axis=0, keepdims=True)
    mom_ref[1:2, :] += jnp.sum(y * y, axis=0, keepdims=True)


def _post(raw, s, b, K, C):
    _, N, Wc = raw.shape
    F = K * Wc
    R = 1000
    return pl.pallas_call(
        functools.partial(_post_body, K=K, C=C),
        grid=(N // R,),
        in_specs=[
            pl.BlockSpec((K, R, Wc), lambda i: (0, i, 0)),
            pl.BlockSpec((2, R, 16), lambda i: (0, i, 0)),
            pl.BlockSpec((1, F), lambda i: (0, 0)),
        ],
        out_specs=[
            pl.BlockSpec((R, F), lambda i: (i, 0)),
            pl.BlockSpec((2, F), lambda i: (0, 0)),
        ],
        out_shape=[
            jax.ShapeDtypeStruct((N, F), jnp.float32),
            jax.ShapeDtypeStruct((2, F), jnp.float32),
        ],
    )(raw, s.reshape(2, N, 16), b.reshape(1, F))


def _pool_body(y_ref, bid_ref, mom_ref, g_ref, be_ref, out_ref, *, n):
    @pl.when(pl.program_id(0) == 0)
    def _():
        out_ref[...] = jnp.full_like(out_ref[...], -jnp.inf)

    mu = mom_ref[0:1, :] / n
    var = mom_ref[1:2, :] / n - mu * mu
    y = (y_ref[...] - mu) * lax.rsqrt(var + 1e-5) * g_ref[...] + be_ref[...]
    y = jnp.maximum(y, 0.0)
    bid = bid_ref[...]
    lo = jnp.min(bid)
    hi = jnp.max(bid)

    def body(b, _):
        m = bid == b
        cm = jnp.max(jnp.where(m, y, -jnp.inf), axis=0, keepdims=True)
        out_ref[pl.ds(b, 1), :] = jnp.maximum(out_ref[pl.ds(b, 1), :], cm)
        return 0

    lax.fori_loop(lo, hi + 1, body, 0)


def _pool(y, bid, mom, g, be):
    N, F = y.shape
    R = 200
    return pl.pallas_call(
        functools.partial(_pool_body, n=N),
        grid=(N // R,),
        in_specs=[
            pl.BlockSpec((R, F), lambda i: (i, 0)),
            pl.BlockSpec((R, 1), lambda i: (i, 0)),
            pl.BlockSpec((2, F), lambda i: (0, 0)),
            pl.BlockSpec((1, F), lambda i: (0, 0)),
            pl.BlockSpec((1, F), lambda i: (0, 0)),
        ],
        out_specs=pl.BlockSpec((64, F), lambda i: (0, 0)),
        out_shape=jax.ShapeDtypeStruct((64, F), jnp.float32),
    )(y, bid, mom, g.reshape(1, F), be.reshape(1, F))


def _fc_body(p_ref, w_ref, b_ref, o_ref):
    p = p_ref[...]
    p = jnp.where(jnp.isfinite(p), p, 0.0)
    o_ref[...] = jnp.dot(p, w_ref[...], preferred_element_type=jnp.float32) + b_ref[...]


def _fc(pooled, Wfc, bfc):
    return pl.pallas_call(
        _fc_body,
        out_shape=jax.ShapeDtypeStruct((pooled.shape[0], Wfc.shape[1]), jnp.float32),
    )(pooled, Wfc, bfc.reshape(1, -1))


def _edge(src, dst, asd_s, asd_d):
    E = src.shape[0]
    B = 128
    NB = (E // B) // (NC * NS)
    XTRA = (E // B) % (NC * NS)

    ebuf_types = [
        pltpu.VMEM((B,), jnp.int32),
        pltpu.VMEM((B,), jnp.int32),
        pltpu.VMEM((B, 16), jnp.float32),
        pltpu.VMEM((B, 16), jnp.float32),
        pltpu.VMEM((B * 8 + 16,), jnp.float32),
        pltpu.SemaphoreType.DMA,
        pltpu.SemaphoreType.DMA,
        pltpu.SemaphoreType.DMA,
    ]

    @functools.partial(
        pl.kernel,
        out_type=[
            jax.ShapeDtypeStruct((E * 8,), jnp.float32),
            jax.ShapeDtypeStruct((2, 16), jnp.float32),
        ],
        mesh=_mesh(),
        scratch_types=ebuf_types + ebuf_types + [
            pltpu.VMEM((NS, 16), jnp.float32),
            pltpu.VMEM((16,), jnp.float32),
            pltpu.VMEM_SHARED((NS, 16), jnp.float32),
        ],
        compiler_params=_SC_PARAMS,
    )
    def k(src_h, dst_h, as_h, ad_h, e_h, m2_h, *refs):
        buf0 = refs[0:8]
        buf1 = refs[8:16]
        red, mxb, shmax = refs[16:19]
        c = lax.axis_index("c")
        s = lax.axis_index("s")
        wid = c * NS + s
        lane = lax.iota(jnp.int32, L)
        neg = jnp.zeros((L,), jnp.float32) - jnp.inf
        base = wid * NB * B

        def off_of(t):
            return pl.multiple_of(base + t * B, 8)

        def issue_small(t, bf):
            off = off_of(t)
            pltpu.async_copy(src_h.at[pl.ds(off, B)], bf[0], bf[5])
            pltpu.async_copy(dst_h.at[pl.ds(off, B)], bf[1], bf[5])

        def wait_small(t, bf):
            off = off_of(t)
            pltpu.make_async_copy(src_h.at[pl.ds(off, B)], bf[0], bf[5]).wait()
            pltpu.make_async_copy(dst_h.at[pl.ds(off, B)], bf[1], bf[5]).wait()

        def issue_gather(bf):
            pltpu.async_copy(as_h.at[bf[0]], bf[2], bf[6])
            pltpu.async_copy(ad_h.at[bf[1]], bf[3], bf[6])

        def wait_gather(bf):
            pltpu.make_async_copy(as_h.at[bf[0]], bf[2], bf[6]).wait()
            pltpu.make_async_copy(ad_h.at[bf[1]], bf[3], bf[6]).wait()

        def compute(bf, mx):
            srows, drows, eblk = bf[2], bf[3], bf[4]
            for r in range(B):
                v = srows[r] + drows[r]
                e16 = jnp.maximum(v, 0.2 * v)
                eblk[pl.ds(r * 8, 16)] = e16
                mx = jnp.maximum(mx, jnp.where(lane < 8, e16, neg))
            return mx

        def estore(t, bf):
            pltpu.async_copy(bf[4].at[pl.ds(0, B * 8)],
                             e_h.at[pl.ds(off_of(t) * 8, B * 8)], bf[7])

        def wait_estore(t, bf):
            pltpu.make_async_copy(bf[4].at[pl.ds(0, B * 8)],
                                  e_h.at[pl.ds(off_of(t) * 8, B * 8)], bf[7]).wait()

        def step(t, cur, nxt, mx):
            wait_small(t + 1, nxt)

            @pl.when(t >= 1)
            def _():
                wait_estore(t - 1, nxt)

            issue_gather(nxt)
            wait_gather(cur)
            mx = compute(cur, mx)
            estore(t, cur)

            @pl.when(t + 2 <= NB - 1)
            def _():
                issue_small(t + 2, cur)

            return mx

        issue_small(0, buf0)
        issue_small(1, buf1)
        wait_small(0, buf0)
        issue_gather(buf0)

        def pair(q, mx):
            t = q * 2
            mx = step(t, buf0, buf1, mx)
            mx = step(t + 1, buf1, buf0, mx)
            return mx

        mx = lax.fori_loop(0, (NB - 1) // 2, pair, neg)
        wait_estore(NB - 2, buf1)
        wait_gather(buf0)
        mx = compute(buf0, mx)
        pltpu.sync_copy(buf0[4].at[pl.ds(0, B * 8)],
                        e_h.at[pl.ds(off_of(NB - 1) * 8, B * 8)])
        offx = pl.multiple_of(
            (NC * NS * NB + jnp.minimum(wid, XTRA - 1)) * B, 8)
        pltpu.sync_copy(src_h.at[pl.ds(offx, B)], buf0[0])
        pltpu.sync_copy(dst_h.at[pl.ds(offx, B)], buf0[1])
        issue_gather(buf0)
        wait_gather(buf0)
        mx = compute(buf0, mx)

        @pl.when(wid < XTRA)
        def _():
            pltpu.sync_copy(buf0[4].at[pl.ds(0, B * 8)],
                            e_h.at[pl.ds(offx * 8, B * 8)])

        mxb[...] = mx
        pltpu.sync_copy(mxb, shmax.at[s])
        plsc.subcore_barrier()

        @pl.when(s == 0)
        def _():
            pltpu.sync_copy(shmax, red)
            m = red[0]
            for t in range(1, NS):
                m = jnp.maximum(m, red[t])
            mxb[...] = m
            pltpu.sync_copy(mxb, m2_h.at[c])

    return k(src, dst, asd_s, asd_d)


def _agg(src, dst, h3, e1, m2, N, K, C):
    E = src.shape[0]
    Wc = h3.shape[1]
    hpc = Wc // C
    Kp = K // 2
    B = 128
    NB = (E // B) // NS
    XTRA = (E // B) % NS
    RPT = N // NS
    NG = B // L
    zW = jnp.zeros((RPT, Wc), jnp.float32)
    zS = jnp.zeros((RPT, 16), jnp.float32)

    buf_types = [
        pltpu.VMEM((B,), jnp.int32),
        pltpu.VMEM((B,), jnp.int32),
        pltpu.VMEM((B,), jnp.int32),
        pltpu.VMEM((B,), jnp.int32),
        pltpu.VMEM((B, Wc), jnp.float32),
        pltpu.VMEM((B * 8 + 16,), jnp.float32),
        pltpu.SemaphoreType.DMA,
        pltpu.SemaphoreType.DMA,
        pltpu.SemaphoreType.DMA,
    ]

    @functools.partial(
        pl.kernel,
        out_type=[
            jax.ShapeDtypeStruct((K * N, Wc), jnp.float32),
            jax.ShapeDtypeStruct((2 * N, 16), jnp.float32),
        ],
        mesh=_mesh(),
        scratch_types=buf_types + buf_types + [
            pltpu.VMEM((B, 16), jnp.float32),
            pltpu.VMEM((2, 16), jnp.float32),
            pltpu.VMEM_SHARED((N, Wc), jnp.float32),
            pltpu.VMEM_SHARED((N, 16), jnp.float32),
        ],
        compiler_params=_SC_PARAMS,
    )
    def k(src_h, dst_h, h3_h, e_h, m2_h, zW_h, zS_h, raw_h, s_h, *refs):
        buf0 = refs[0:9]
        buf1 = refs[9:18]
        exb, m2v, acc, sacc = refs[18:22]
        c = lax.axis_index("c")
        s = lax.axis_index("s")
        lane = lax.iota(jnp.int32, L)
        pltpu.sync_copy(m2_h, m2v)
        mv16 = jnp.maximum(m2v[0], m2v[1])
        l8 = lane & 7
        mpat = jnp.zeros((L,), jnp.float32)
        for hh in range(8):
            mpat = jnp.where(l8 == hh, mv16[hh], mpat)
        base = s * NB * B

        def off_of(t):
            return pl.multiple_of(base + t * B, 8)

        def issue_small(t, bf):
            off = off_of(t)
            pltpu.async_copy(src_h.at[pl.ds(off, B)], bf[0], bf[6])
            pltpu.async_copy(dst_h.at[pl.ds(off, B)], bf[1], bf[6])
            pltpu.async_copy(e_h.at[pl.ds(off * 8, B * 8)],
                             bf[5].at[pl.ds(0, B * 8)], bf[6])

        def wait_small(t, bf):
            off = off_of(t)
            pltpu.make_async_copy(src_h.at[pl.ds(off, B)], bf[0], bf[6]).wait()
            pltpu.make_async_copy(dst_h.at[pl.ds(off, B)], bf[1], bf[6]).wait()
            pltpu.make_async_copy(e_h.at[pl.ds(off * 8, B * 8)],
                                  bf[5].at[pl.ds(0, B * 8)], bf[6]).wait()

        def compute(t, bf, kk, chunk, mh_l):
            dstv, dscv, rows, ev = bf[1], bf[2], bf[4], bf[5]

            def cpi(i, _):
                dscv[pl.ds(i * L, L)] = dstv[pl.ds(i * L, L)]
                return 0

            lax.fori_loop(0, B // L, cpi, 0)

            def grp(g, _):
                rvec = g * L + lane
                exs_l = [
                    jnp.exp(plsc.load_gather(ev, [rvec * 8 + (chunk * hpc + hh)])
                            - mh_l[hh])
                    for hh in range(hpc)
                ]
                for r16 in range(L):
                    rsp = jnp.zeros((L,), jnp.int32) + (g * L + r16)
                    for hh in range(hpc):
                        exs = exs_l[hh][r16]
                        for j in range(C // L):
                            cv = lane + (hh * C + j * L)
                            v = plsc.load_gather(rows, [rsp, cv])
                            plsc.store_scatter(rows, [rsp, cv], v * exs)
                return 0

            lax.fori_loop(0, NG, grp, 0)

            if kk == 0:
                @pl.when(jnp.where(c == 0, t < NB // 2, t >= NB // 2))
                def _():
                    def srow(r, _):
                        rsp = jnp.zeros((L,), jnp.int32) + r
                        v = ev[pl.ds(r * 8, 16)]
                        plsc.store_scatter(exb, [rsp, lane],
                                           jnp.where(lane < 8, jnp.exp(v - mpat), 0.0))
                        return 0

                    lax.fori_loop(0, B, srow, 0)
                    pltpu.sync_copy(exb, sacc.at[dscv], add=True)

        def mk_idx(t, bf, chunk):
            srcv, idxv = bf[0], bf[3]

            def idxb(i, _):
                idxv[pl.ds(i * L, L)] = srcv[pl.ds(i * L, L)] + chunk * N
                return 0

            lax.fori_loop(0, B // L, idxb, 0)

        def step(t, cur, nxt, kk, chunk, mh_l):
            wait_small(t + 1, nxt)
            mk_idx(t + 1, nxt, chunk)

            @pl.when(t >= 1)
            def _():
                pltpu.make_async_copy(nxt[4], acc.at[nxt[2]], nxt[8]).wait()

            pltpu.async_copy(h3_h.at[nxt[3]], nxt[4], nxt[7])
            pltpu.make_async_copy(h3_h.at[cur[3]], cur[4], cur[7]).wait()
            compute(t, cur, kk, chunk, mh_l)
            pltpu.async_copy(cur[4], acc.at[cur[2]], cur[8], add=True)

            @pl.when(t + 2 <= NB - 1)
            def _():
                issue_small(t + 2, cur)

        for kk in range(Kp):
            chunk = c * Kp + kk
            mh_l = [jnp.sum(jnp.where(lane == chunk * hpc + hh, mv16, 0.0))
                    for hh in range(hpc)]
            pltpu.sync_copy(zW_h, acc.at[pl.ds(s * RPT, RPT)])
            if kk == 0:
                pltpu.sync_copy(zS_h, sacc.at[pl.ds(s * RPT, RPT)])
            plsc.subcore_barrier()

            issue_small(0, buf0)
            issue_small(1, buf1)
            wait_small(0, buf0)
            mk_idx(0, buf0, chunk)
            pltpu.async_copy(h3_h.at[buf0[3]], buf0[4], buf0[7])

            def pair(q, _):
                t = q * 2
                step(t, buf0, buf1, kk, chunk, mh_l)
                step(t + 1, buf1, buf0, kk, chunk, mh_l)
                return 0

            lax.fori_loop(0, (NB - 2) // 2, pair, 0)
            step(jnp.int32(NB - 2), buf0, buf1, kk, chunk, mh_l)
            pltpu.make_async_copy(buf0[4], acc.at[buf0[2]], buf0[8]).wait()
            pltpu.make_async_copy(h3_h.at[buf1[3]], buf1[4], buf1[7]).wait()
            compute(jnp.int32(NB - 1), buf1, kk, chunk, mh_l)
            pltpu.sync_copy(buf1[4], acc.at[buf1[2]], add=True)

            @pl.when(s < XTRA)
            def _():
                offx = pl.multiple_of((NS * NB + s) * B, 8)
                pltpu.sync_copy(src_h.at[pl.ds(offx, B)], buf0[0])
                pltpu.sync_copy(dst_h.at[pl.ds(offx, B)], buf0[1])
                pltpu.sync_copy(e_h.at[pl.ds(offx * 8, B * 8)],
                                buf0[5].at[pl.ds(0, B * 8)])
                mk_idx(0, buf0, chunk)
                pltpu.async_copy(h3_h.at[buf0[3]], buf0[4], buf0[7]).wait()
                compute(jnp.int32(0), buf0, kk, chunk, mh_l)
                pltpu.sync_copy(buf0[4], acc.at[buf0[2]], add=True)

            plsc.subcore_barrier()
            pltpu.sync_copy(acc.at[pl.ds(s * RPT, RPT)],
                            raw_h.at[pl.ds(chunk * N + s * RPT, RPT)])
            if kk == 0:
                pltpu.sync_copy(sacc.at[pl.ds(s * RPT, RPT)],
                                s_h.at[pl.ds(c * N + s * RPT, RPT)])
            plsc.subcore_barrier()

    return k(src, dst, h3, e1, m2, zW, zS)


def _att_w(att_s, att_d, F, C):
    hid = jnp.repeat(jnp.arange(8), C)
    onehot = (hid[:, None] == jnp.arange(8)[None, :]).astype(jnp.float32)
    AS = att_s.reshape(F, 1) * onehot
    AD = att_d.reshape(F, 1) * onehot
    z = jnp.zeros((F, 8), jnp.float32)
    return jnp.concatenate([AS, z, AD, z], axis=1)


def _layer(src, dst, xin, W, aw, mom, g, be, b, K, C, bn):
    N = xin.shape[0]
    h3, as_t, ad_t = _dense(xin, W, aw, mom, g, be, K=K, bn=bn)
    Wc = h3.shape[2]
    e1, m2 = _edge(src, dst, as_t, ad_t)
    raw, s = _agg(src, dst, h3.reshape(K * N, Wc), e1, m2, N, K, C)
    return _post(raw.reshape(K, N, Wc), s, b, K=K, C=C)


def kernel(x, edge_index, batch, W1, as1, ad1, b1, g1, be1, W2, as2, ad2, b2, g2,
           be2, W3, as3, ad3, b3, g3, be3, Wfc, bfc):
    N = x.shape[0]
    src = edge_index[0]
    dst = edge_index[1]
    xp = jnp.pad(x, ((0, 0), (0, 5)))
    W1p = jnp.pad(W1, ((0, 5), (0, 0)))
    aw1 = _att_w(as1, ad1, 128, 16)
    aw2 = _att_w(as2, ad2, 256, 32)
    aw3 = _att_w(as3, ad3, 512, 64)

    y1, mom1 = _layer(src, dst, xp, W1p, aw1, None, None, None, b1, K=2, C=16, bn=False)
    y2, mom2 = _layer(src, dst, y1, W2, aw2, mom1, g1, be1, b2, K=2, C=32, bn=True)
    y3, mom3 = _layer(src, dst, y2, W3, aw3, mom2, g2, be2, b3, K=4, C=64, bn=True)

    pooled = _pool(y3, batch.reshape(N, 1), mom3, g3, be3)
    return _fc(pooled, Wfc, bfc)
